# Initial kernel scaffold; baseline (speedup 1.0000x reference)
import jax
import jax.numpy as jnp
from jax.experimental import pallas as pl


def kernel(x, edge_index, W_l1, b_l1, W_r1, g1, be1, W_l2, b_l2, W_r2, g2, be2, W_g, att_s, att_d, b_g, Wc1, bc1, Wc2, bc2):
    raise NotImplementedError("write your pallas kernel here")


# speedup vs baseline: 29.9358x; 29.9358x over previous
import functools

import jax
import jax.numpy as jnp
from jax import lax
from jax.experimental import pallas as pl
from jax.experimental.pallas import tpu as pltpu
from jax.experimental.pallas import tpu_sc as plsc

N = 10000
E = 320000
D = 128
HEADS = 4
D_OUT = 32

NCORES = 2
NSUB = 16
NTILES = NCORES * NSUB
CHUNK = 128
NCH = 79
EPAD = NTILES * NCH * CHUNK
NPAD = N + 112
RPT = NPAD // NSUB

BLK = 2000
NBLK = N // BLK

_SC_PARAMS = pltpu.CompilerParams(use_tc_tiling_on_sc=False,
                                  needs_layout_passes=False)


@functools.lru_cache(maxsize=None)
def _mesh():
    return plsc.VectorSubcoreMesh(
        core_axis_name="c", subcore_axis_name="s",
        num_cores=NCORES, num_subcores=NSUB)


@functools.lru_cache(maxsize=None)
def _make_sage_sc(width):
    @functools.partial(
        pl.kernel,
        out_type=jax.ShapeDtypeStruct((NCORES, NPAD, width), jnp.float32),
        mesh=_mesh(),
        scratch_types=[
            pltpu.VMEM_SHARED((NPAD, width), jnp.float32),
            pltpu.VMEM((1, CHUNK), jnp.int32),
            pltpu.VMEM((1, CHUNK), jnp.int32),
            pltpu.VMEM((CHUNK, width), jnp.float32),
        ],
        compiler_params=_SC_PARAMS,
    )
    def sage_sc(tab_hbm, src_hbm, dst_hbm, zero_hbm, out_hbm,
                acc, src_scr, dst_scr, buf):
        c = lax.axis_index("c")
        s = lax.axis_index("s")
        wid = c * NSUB + s
        r0 = s * RPT
        pltpu.sync_copy(zero_hbm.at[pl.ds(r0, RPT)], acc.at[pl.ds(r0, RPT)])
        plsc.subcore_barrier()

        @pl.loop(0, NCH)
        def _(j):
            pltpu.sync_copy(src_hbm.at[pl.ds(wid * NCH + j, 1)], src_scr)
            pltpu.sync_copy(dst_hbm.at[pl.ds(wid * NCH + j, 1)], dst_scr)
            pltpu.sync_copy(tab_hbm.at[src_scr.at[0]], buf)
            pltpu.sync_copy(buf, acc.at[dst_scr.at[0]], add=True)

        plsc.subcore_barrier()
        pltpu.sync_copy(acc.at[pl.ds(r0, RPT)], out_hbm.at[c, pl.ds(r0, RPT)])

    return sage_sc


@functools.lru_cache(maxsize=None)
def _make_gat_sc():
  @functools.partial(
    pl.kernel,
    out_type=(jax.ShapeDtypeStruct((NCORES, NPAD, D), jnp.float32),
              jax.ShapeDtypeStruct((NCORES, NPAD, 16), jnp.float32)),
    mesh=_mesh(),
    scratch_types=[
        pltpu.VMEM_SHARED((NPAD, D), jnp.float32),
        pltpu.VMEM_SHARED((NPAD, 16), jnp.float32),
        pltpu.VMEM((1, CHUNK), jnp.int32),
        pltpu.VMEM((1, CHUNK), jnp.int32),
        pltpu.VMEM((CHUNK, 16), jnp.float32),
        pltpu.VMEM((CHUNK, 16), jnp.float32),
        pltpu.VMEM((CHUNK, 16), jnp.float32),
        pltpu.VMEM((CHUNK, D), jnp.float32),
        pltpu.VMEM((2 * NBLK, 16), jnp.float32),
    ],
    compiler_params=_SC_PARAMS,
  )
  def _gat_sc(hg_hbm, tabs_hbm, tabd_hbm, cs_hbm, src_hbm, dst_hbm,
              z128_hbm, z16_hbm, num_hbm, den_hbm,
              numacc, denacc, src_scr, dst_scr, sbuf, dbuf, wst, hbuf,
              cscr):
    c = lax.axis_index("c")
    s = lax.axis_index("s")
    wid = c * NSUB + s
    r0 = s * RPT
    pltpu.sync_copy(z128_hbm.at[pl.ds(r0, RPT)], numacc.at[pl.ds(r0, RPT)])
    pltpu.sync_copy(z16_hbm.at[pl.ds(r0, RPT)], denacc.at[pl.ds(r0, RPT)])
    pltpu.sync_copy(cs_hbm, cscr)
    plsc.subcore_barrier()
    ms = cscr[0, :]
    md = cscr[1, :]
    for b in range(1, NBLK):
        ms = jnp.maximum(ms, cscr[2 * b, :])
        md = jnp.maximum(md, cscr[2 * b + 1, :])
    cvec = jnp.maximum(ms + md, 0.0)

    @pl.loop(0, NCH)
    def _(j):
        pltpu.sync_copy(src_hbm.at[pl.ds(wid * NCH + j, 1)], src_scr)
        pltpu.sync_copy(dst_hbm.at[pl.ds(wid * NCH + j, 1)], dst_scr)
        pltpu.sync_copy(tabs_hbm.at[src_scr.at[0]], sbuf)
        pltpu.sync_copy(tabd_hbm.at[dst_scr.at[0]], dbuf)
        pltpu.sync_copy(hg_hbm.at[src_scr.at[0]], hbuf)

        @pl.loop(0, CHUNK)
        def _(i):
            a = sbuf[i, :] + dbuf[i, :]
            w = jnp.exp(jnp.maximum(a, 0.2 * a) - cvec)
            wst[i, :] = w
            for head in range(HEADS):
                wb = plsc.load_gather(
                    wst, [jnp.full((16,), i, jnp.int32),
                          jnp.full((16,), head, jnp.int32)])
                for half in range(2):
                    col = (head * 2 + half) * 16
                    hbuf[i, pl.ds(col, 16)] = hbuf[i, pl.ds(col, 16)] * wb

        pltpu.sync_copy(wst, denacc.at[dst_scr.at[0]], add=True)
        pltpu.sync_copy(hbuf, numacc.at[dst_scr.at[0]], add=True)

    plsc.subcore_barrier()
    pltpu.sync_copy(numacc.at[pl.ds(r0, RPT)], num_hbm.at[c, pl.ds(r0, RPT)])
    pltpu.sync_copy(denacc.at[pl.ds(r0, RPT)], den_hbm.at[c, pl.ds(r0, RPT)])

  return _gat_sc


def _blk(shape):
    nd = len(shape)
    return pl.BlockSpec((BLK,) + shape[1:], lambda i: (i,) + (0,) * (nd - 1))


def _full(shape):
    nd = len(shape)
    return pl.BlockSpec(shape, lambda i: (0,) * nd)


def _sage_dense_body(a0, a1, xr, wl, bl, wr, hpre_out, stats_out, invc_out):
    su = a0[:, :D] + a1[:, :D]
    cnt = a0[:, D:D + 1] + a1[:, D:D + 1]
    invc = 1.0 / jnp.maximum(cnt, 1.0)
    mean = su * invc
    h = (jnp.dot(mean, wl[...], preferred_element_type=jnp.float32) + bl[...]
         + jnp.dot(xr[...], wr[...], preferred_element_type=jnp.float32))
    hpre_out[...] = h
    stats_out[0, 0, :] = jnp.sum(h, axis=0)
    stats_out[0, 1, :] = jnp.sum(h * h, axis=0)
    invc_out[...] = jnp.broadcast_to(invc, (BLK, 8))


def _sage_dense(a0, a1, xr, wl, bl, wr, width):
    return pl.pallas_call(
        _sage_dense_body,
        grid=(NBLK,),
        in_specs=[_blk((BLK, width)), _blk((BLK, width)), _blk((BLK, D)),
                  _full((D, D)), _full((1, D)), _full((D, D))],
        out_specs=(_blk((BLK, D)),
                   pl.BlockSpec((1, 2, D), lambda i: (i, 0, 0)),
                   _blk((BLK, 8))),
        out_shape=(jax.ShapeDtypeStruct((N, D), jnp.float32),
                   jax.ShapeDtypeStruct((NBLK, 2, D), jnp.float32),
                   jax.ShapeDtypeStruct((N, 8), jnp.float32)),
    )(a0, a1, xr, wl, bl, wr)


def _sage_dense2_body(a0, a1, invc, xr, wl, bl, wr, hpre_out, stats_out):
    su = a0[...] + a1[...]
    mean = su * invc[:, 0:1]
    h = (jnp.dot(mean, wl[...], preferred_element_type=jnp.float32) + bl[...]
         + jnp.dot(xr[...], wr[...], preferred_element_type=jnp.float32))
    hpre_out[...] = h
    stats_out[0, 0, :] = jnp.sum(h, axis=0)
    stats_out[0, 1, :] = jnp.sum(h * h, axis=0)


def _sage_dense2(a0, a1, invc, xr, wl, bl, wr):
    return pl.pallas_call(
        _sage_dense2_body,
        grid=(NBLK,),
        in_specs=[_blk((BLK, D)), _blk((BLK, D)), _blk((BLK, 8)),
                  _blk((BLK, D)), _full((D, D)), _full((1, D)),
                  _full((D, D))],
        out_specs=(_blk((BLK, D)),
                   pl.BlockSpec((1, 2, D), lambda i: (i, 0, 0))),
        out_shape=(jax.ShapeDtypeStruct((N, D), jnp.float32),
                   jax.ShapeDtypeStruct((NBLK, 2, D), jnp.float32)),
    )(a0, a1, invc, xr, wl, bl, wr)


def _bn_finish_body(hpre, stats, resid, g, be, h_out):
    mu = jnp.sum(stats[:, 0, :], axis=0, keepdims=True) * (1.0 / N)
    ex2 = jnp.sum(stats[:, 1, :], axis=0, keepdims=True) * (1.0 / N)
    var = ex2 - mu * mu
    h = g[...] * (hpre[...] - mu) * jax.lax.rsqrt(var + 1e-5) + be[...]
    h_out[...] = jnp.maximum(h + resid[...], 0.0)


def _bn_finish(hpre, stats, resid, g, be):
    return pl.pallas_call(
        _bn_finish_body,
        grid=(NBLK,),
        in_specs=[_blk((BLK, D)), _full((NBLK, 2, D)), _blk((BLK, D)),
                  _full((1, D)), _full((1, D))],
        out_specs=_blk((BLK, D)),
        out_shape=jax.ShapeDtypeStruct((N, D), jnp.float32),
    )(hpre, stats, resid, g, be)


def _bn_gat_body(hpre, stats, resid, g, be, wg, as16, ad16,
                 hg_out, ts_out, td_out, cs_out):
    mu = jnp.sum(stats[:, 0, :], axis=0, keepdims=True) * (1.0 / N)
    ex2 = jnp.sum(stats[:, 1, :], axis=0, keepdims=True) * (1.0 / N)
    var = ex2 - mu * mu
    h = g[...] * (hpre[...] - mu) * jax.lax.rsqrt(var + 1e-5) + be[...]
    h2 = jnp.maximum(h + resid[...], 0.0)
    hg = jnp.dot(h2, wg[...], preferred_element_type=jnp.float32)
    a_s = jnp.dot(hg, as16[...], preferred_element_type=jnp.float32)
    a_d = jnp.dot(hg, ad16[...], preferred_element_type=jnp.float32)
    hg_out[...] = hg
    ts_out[...] = a_s
    td_out[...] = a_d
    cs_out[0, 0, :] = jnp.max(a_s, axis=0)
    cs_out[0, 1, :] = jnp.max(a_d, axis=0)


def _bn_gat(hpre, stats, resid, g, be, wg, as16, ad16):
    return pl.pallas_call(
        _bn_gat_body,
        grid=(NBLK,),
        in_specs=[_blk((BLK, D)), _full((NBLK, 2, D)), _blk((BLK, D)),
                  _full((1, D)), _full((1, D)), _full((D, D)),
                  _full((D, 16)), _full((D, 16))],
        out_specs=(_blk((BLK, D)), _blk((BLK, 16)), _blk((BLK, 16)),
                   pl.BlockSpec((1, 2, 16), lambda i: (i, 0, 0))),
        out_shape=(jax.ShapeDtypeStruct((N, D), jnp.float32),
                   jax.ShapeDtypeStruct((N, 16), jnp.float32),
                   jax.ShapeDtypeStruct((N, 16), jnp.float32),
                   jax.ShapeDtypeStruct((NBLK, 2, 16), jnp.float32)),
    )(hpre, stats, resid, g, be, wg, as16, ad16)


def _head_body(n0, n1, d0, d1, hg, ts, td, cs, bg, wc1, bc1, wc2, bc2,
               emb_out, log_out):
    ms = jnp.max(cs[:, 0, :], axis=0, keepdims=True)
    md = jnp.max(cs[:, 1, :], axis=0, keepdims=True)
    c16 = jnp.maximum(ms + md, 0.0)
    asum = ts[...] + td[...]
    wself = jnp.exp(jnp.maximum(asum, 0.2 * asum) - c16)
    den = d0[...] + d1[...] + wself
    num = n0[...] + n1[...]
    emb = jnp.zeros((BLK, D_OUT), jnp.float32)
    for head in range(HEADS):
        blk = (num[:, head * D_OUT:(head + 1) * D_OUT]
               + hg[:, head * D_OUT:(head + 1) * D_OUT]
               * wself[:, head:head + 1])
        emb = emb + blk / (den[:, head:head + 1] + 1e-16)
    emb = emb * (1.0 / HEADS) + bg[...]
    z = jnp.maximum(
        jnp.dot(emb, wc1[...], preferred_element_type=jnp.float32) + bc1[...],
        0.0)
    log_out[...] = (jnp.dot(z, wc2[...], preferred_element_type=jnp.float32)
                    + bc2[...])
    emb_out[...] = emb


def _head(n0, n1, d0, d1, hg, ts, td, cs, bg, wc1, bc1, wc2, bc2):
    return pl.pallas_call(
        _head_body,
        grid=(NBLK,),
        in_specs=[_blk((BLK, D)), _blk((BLK, D)), _blk((BLK, 16)),
                  _blk((BLK, 16)), _blk((BLK, D)), _blk((BLK, 16)),
                  _blk((BLK, 16)), _full((NBLK, 2, 16)), _full((1, D_OUT)),
                  _full((D_OUT, 64)), _full((1, 64)), _full((64, 1)),
                  _full((1, 1))],
        out_specs=(_blk((BLK, D_OUT)), _blk((BLK, 1))),
        out_shape=(jax.ShapeDtypeStruct((N, D_OUT), jnp.float32),
                   jax.ShapeDtypeStruct((N, 1), jnp.float32)),
    )(n0, n1, d0, d1, hg, ts, td, cs, bg, wc1, bc1, wc2, bc2)


def _pad_rows(a):
    return jnp.concatenate(
        [a, jnp.zeros((NPAD - N, a.shape[1]), a.dtype)], axis=0)


def kernel(x, edge_index, W_l1, b_l1, W_r1, g1, be1, W_l2, b_l2, W_r2, g2,
           be2, W_g, att_s, att_d, b_g, Wc1, bc1, Wc2, bc2):
    pad = N + (jnp.arange(EPAD - E, dtype=jnp.int32) % 32)
    src = jnp.concatenate([edge_index[0], pad]).reshape(NTILES * NCH, CHUNK)
    dst = jnp.concatenate([edge_index[1], pad]).reshape(NTILES * NCH, CHUNK)

    xa = jnp.zeros((NPAD, D + 16), jnp.float32)
    xa = xa.at[:N, :D].set(x).at[:N, D].set(1.0)

    rows = jnp.arange(D)
    cols = jnp.repeat(jnp.arange(HEADS), D_OUT)
    as16 = jnp.zeros((D, 16), jnp.float32).at[rows, cols].set(att_s.reshape(-1))
    ad16 = jnp.zeros((D, 16), jnp.float32).at[rows, cols].set(att_d.reshape(-1))

    z144 = jnp.zeros((NPAD, D + 16), jnp.float32)
    z128 = jnp.zeros((NPAD, D), jnp.float32)
    z16 = jnp.zeros((NPAD, 16), jnp.float32)

    b_l1r = b_l1.reshape(1, D)
    g1r = g1.reshape(1, D)
    be1r = be1.reshape(1, D)
    b_l2r = b_l2.reshape(1, D)
    g2r = g2.reshape(1, D)
    be2r = be2.reshape(1, D)
    bgr = b_g.reshape(1, D_OUT)
    bc1r = bc1.reshape(1, 64)
    bc2r = bc2.reshape(1, 1)

    acc1 = _make_sage_sc(D + 16)(xa, src, dst, z144)
    hpre1, st1, invc = _sage_dense(acc1[0], acc1[1], x, W_l1, b_l1r, W_r1,
                                   D + 16)
    h = _bn_finish(hpre1, st1, x, g1r, be1r)

    acc2 = _make_sage_sc(D)(_pad_rows(h), src, dst, z128)
    hpre2, st2 = _sage_dense2(acc2[0], acc2[1], invc, h, W_l2, b_l2r, W_r2)
    hg, ts, td, cs = _bn_gat(hpre2, st2, h, g2r, be2r, W_g, as16, ad16)

    num, den = _make_gat_sc()(_pad_rows(hg), _pad_rows(ts), _pad_rows(td),
                              cs.reshape(2 * NBLK, 16), src, dst, z128, z16)
    emb, logits = _head(num[0], num[1], den[0], den[1], hg, ts, td, cs,
                        bgr, Wc1, bc1r, Wc2, bc2r)
    return (emb, logits)


# speedup vs baseline: 42.0000x; 1.4030x over previous
import functools

import jax
import jax.numpy as jnp
from jax import lax
from jax.experimental import pallas as pl
from jax.experimental.pallas import tpu as pltpu
from jax.experimental.pallas import tpu_sc as plsc

N = 10000
E = 320000
D = 128
DW = D + 16
HEADS = 4
D_OUT = 32

NCORES = 2
NSUB = 16
NTILES = NCORES * NSUB
CHUNK = 128
NCH = 80
NPAIR = NCH // 2
EPAD = NTILES * NCH * CHUNK
NPAD = N + 16
RPT = NPAD // NSUB

BLK = 2000
NBLK = N // BLK

_SC_PARAMS = pltpu.CompilerParams(use_tc_tiling_on_sc=False,
                                  needs_layout_passes=False)


@functools.lru_cache(maxsize=None)
def _mesh():
    return plsc.VectorSubcoreMesh(
        core_axis_name="c", subcore_axis_name="s",
        num_cores=NCORES, num_subcores=NSUB)


def _wait(src, dst, sem):
    pltpu.make_async_copy(src, dst, sem).wait()


@functools.lru_cache(maxsize=None)
def _make_sage_sc(width):
    @functools.partial(
        pl.kernel,
        out_type=jax.ShapeDtypeStruct((NCORES, NPAD, width), jnp.float32),
        mesh=_mesh(),
        scratch_types=[
            pltpu.VMEM_SHARED((NPAD, width), jnp.float32),
            pltpu.VMEM((2, CHUNK), jnp.int32),
            pltpu.VMEM((2, CHUNK), jnp.int32),
            pltpu.VMEM((CHUNK, width), jnp.float32),
            pltpu.VMEM((CHUNK, width), jnp.float32),
            pltpu.SemaphoreType.DMA,
            pltpu.SemaphoreType.DMA,
            pltpu.SemaphoreType.DMA,
            pltpu.SemaphoreType.DMA,
        ],
        compiler_params=_SC_PARAMS,
    )
    def sage_sc(tab_hbm, idx_hbm, zero_hbm, out_hbm,
                acc, ixa, ixb, bufa, bufb, sga, sgb, ssa, ssb):
        c = lax.axis_index("c")
        s = lax.axis_index("s")
        base = (c * NSUB + s) * NCH
        r0 = s * RPT
        pltpu.sync_copy(zero_hbm.at[pl.ds(r0, RPT)], acc.at[pl.ds(r0, RPT)])
        plsc.subcore_barrier()

        pltpu.sync_copy(idx_hbm.at[base], ixa)
        pltpu.sync_copy(idx_hbm.at[base + 1], ixb)
        pltpu.async_copy(tab_hbm.at[ixa.at[0]], bufa, sga)
        pltpu.async_copy(tab_hbm.at[ixb.at[0]], bufb, sgb)

        @pl.loop(0, NPAIR - 1)
        def _(k):
            _wait(tab_hbm.at[ixa.at[0]], bufa, sga)
            pltpu.async_copy(bufa, acc.at[ixa.at[1]], ssa, add=True)
            _wait(tab_hbm.at[ixb.at[0]], bufb, sgb)
            pltpu.async_copy(bufb, acc.at[ixb.at[1]], ssb, add=True)
            _wait(bufa, acc.at[ixa.at[1]], ssa)
            pltpu.sync_copy(idx_hbm.at[base + 2 * k + 2], ixa)
            pltpu.async_copy(tab_hbm.at[ixa.at[0]], bufa, sga)
            _wait(bufb, acc.at[ixb.at[1]], ssb)
            pltpu.sync_copy(idx_hbm.at[base + 2 * k + 3], ixb)
            pltpu.async_copy(tab_hbm.at[ixb.at[0]], bufb, sgb)

        _wait(tab_hbm.at[ixa.at[0]], bufa, sga)
        pltpu.async_copy(bufa, acc.at[ixa.at[1]], ssa, add=True)
        _wait(tab_hbm.at[ixb.at[0]], bufb, sgb)
        pltpu.async_copy(bufb, acc.at[ixb.at[1]], ssb, add=True)
        _wait(bufa, acc.at[ixa.at[1]], ssa)
        _wait(bufb, acc.at[ixb.at[1]], ssb)

        plsc.subcore_barrier()
        pltpu.sync_copy(acc.at[pl.ds(r0, RPT)], out_hbm.at[c, pl.ds(r0, RPT)])

    return sage_sc


@functools.lru_cache(maxsize=None)
def _make_gat_sc():
  @functools.partial(
    pl.kernel,
    out_type=jax.ShapeDtypeStruct((NCORES, NPAD, DW), jnp.float32),
    mesh=_mesh(),
    scratch_types=[
        pltpu.VMEM_SHARED((NPAD, DW), jnp.float32),
        pltpu.VMEM((2, CHUNK), jnp.int32),
        pltpu.VMEM((2, CHUNK), jnp.int32),
        pltpu.VMEM((CHUNK, DW), jnp.float32),
        pltpu.VMEM((CHUNK, DW), jnp.float32),
        pltpu.VMEM((CHUNK, 16), jnp.float32),
        pltpu.VMEM((2 * NBLK, 16), jnp.float32),
        pltpu.SemaphoreType.DMA,
        pltpu.SemaphoreType.DMA,
        pltpu.SemaphoreType.DMA,
        pltpu.SemaphoreType.DMA,
        pltpu.SemaphoreType.DMA,
    ],
    compiler_params=_SC_PARAMS,
  )
  def _gat_sc(hgs_hbm, tabd_hbm, cs_hbm, idx_hbm, zero_hbm, out_hbm,
              acc, ixa, ixb, ha, hb, dbuf, cscr, sga, sgb, sda, ssa, ssb):
    c = lax.axis_index("c")
    s = lax.axis_index("s")
    base = (c * NSUB + s) * NCH
    r0 = s * RPT
    pltpu.sync_copy(zero_hbm.at[pl.ds(r0, RPT)], acc.at[pl.ds(r0, RPT)])
    pltpu.sync_copy(cs_hbm, cscr)
    plsc.subcore_barrier()
    ms = cscr[0, :]
    md = cscr[1, :]
    for b in range(1, NBLK):
        ms = jnp.maximum(ms, cscr[2 * b, :])
        md = jnp.maximum(md, cscr[2 * b + 1, :])
    cvec = jnp.maximum(ms + md, 0.0)

    def compute(hbuf):
        @pl.loop(0, CHUNK, step=4)
        def _(e0):
            for u in range(4):
                e = e0 + u
                a = hbuf[e, pl.ds(D, 16)] + dbuf[e, :]
                hbuf[e, pl.ds(D, 16)] = jnp.exp(
                    jnp.maximum(a, 0.2 * a) - cvec)

        @pl.loop(0, CHUNK, step=4)
        def _(i0):
            for u in range(4):
                i = i0 + u
                ivec = jnp.full((16,), i, jnp.int32)
                for head in range(HEADS):
                    wb = plsc.load_gather(
                        hbuf, [ivec, jnp.full((16,), D + head, jnp.int32)])
                    for half in range(2):
                        col = (head * 2 + half) * 16
                        hbuf[i, pl.ds(col, 16)] = (
                            hbuf[i, pl.ds(col, 16)] * wb)

    pltpu.sync_copy(idx_hbm.at[base], ixa)
    pltpu.sync_copy(idx_hbm.at[base + 1], ixb)
    pltpu.async_copy(hgs_hbm.at[ixa.at[0]], ha, sga)
    pltpu.async_copy(hgs_hbm.at[ixb.at[0]], hb, sgb)
    pltpu.async_copy(tabd_hbm.at[ixa.at[1]], dbuf, sda)

    @pl.loop(0, NPAIR - 1)
    def _(k):
        _wait(hgs_hbm.at[ixa.at[0]], ha, sga)
        _wait(tabd_hbm.at[ixa.at[1]], dbuf, sda)
        compute(ha)
        pltpu.async_copy(ha, acc.at[ixa.at[1]], ssa, add=True)
        pltpu.async_copy(tabd_hbm.at[ixb.at[1]], dbuf, sda)
        _wait(hgs_hbm.at[ixb.at[0]], hb, sgb)
        _wait(tabd_hbm.at[ixb.at[1]], dbuf, sda)
        compute(hb)
        pltpu.async_copy(hb, acc.at[ixb.at[1]], ssb, add=True)
        _wait(ha, acc.at[ixa.at[1]], ssa)
        pltpu.sync_copy(idx_hbm.at[base + 2 * k + 2], ixa)
        pltpu.async_copy(hgs_hbm.at[ixa.at[0]], ha, sga)
        pltpu.async_copy(tabd_hbm.at[ixa.at[1]], dbuf, sda)
        _wait(hb, acc.at[ixb.at[1]], ssb)
        pltpu.sync_copy(idx_hbm.at[base + 2 * k + 3], ixb)
        pltpu.async_copy(hgs_hbm.at[ixb.at[0]], hb, sgb)

    _wait(hgs_hbm.at[ixa.at[0]], ha, sga)
    _wait(tabd_hbm.at[ixa.at[1]], dbuf, sda)
    compute(ha)
    pltpu.async_copy(ha, acc.at[ixa.at[1]], ssa, add=True)
    pltpu.async_copy(tabd_hbm.at[ixb.at[1]], dbuf, sda)
    _wait(hgs_hbm.at[ixb.at[0]], hb, sgb)
    _wait(tabd_hbm.at[ixb.at[1]], dbuf, sda)
    compute(hb)
    pltpu.async_copy(hb, acc.at[ixb.at[1]], ssb, add=True)
    _wait(ha, acc.at[ixa.at[1]], ssa)
    _wait(hb, acc.at[ixb.at[1]], ssb)

    plsc.subcore_barrier()
    pltpu.sync_copy(acc.at[pl.ds(r0, RPT)], out_hbm.at[c, pl.ds(r0, RPT)])

  return _gat_sc


def _blk(shape):
    nd = len(shape)
    return pl.BlockSpec((BLK,) + shape[1:], lambda i: (i,) + (0,) * (nd - 1))


def _full(shape):
    nd = len(shape)
    return pl.BlockSpec(shape, lambda i: (0,) * nd)


def _sage_dense_body(a0, a1, xr, wl, bl, wr, hpre_out, stats_out, invc_out):
    su = a0[:, :D] + a1[:, :D]
    cnt = a0[:, D:D + 1] + a1[:, D:D + 1]
    invc = 1.0 / jnp.maximum(cnt, 1.0)
    mean = su * invc
    h = (jnp.dot(mean, wl[...], preferred_element_type=jnp.float32) + bl[...]
         + jnp.dot(xr[...], wr[...], preferred_element_type=jnp.float32))
    hpre_out[...] = h
    stats_out[0, 0, :] = jnp.sum(h, axis=0)
    stats_out[0, 1, :] = jnp.sum(h * h, axis=0)
    invc_out[...] = jnp.broadcast_to(invc, (BLK, 8))


def _sage_dense(a0, a1, xr, wl, bl, wr):
    return pl.pallas_call(
        _sage_dense_body,
        grid=(NBLK,),
        in_specs=[_blk((BLK, DW)), _blk((BLK, DW)), _blk((BLK, D)),
                  _full((D, D)), _full((1, D)), _full((D, D))],
        out_specs=(_blk((BLK, D)),
                   pl.BlockSpec((1, 2, D), lambda i: (i, 0, 0)),
                   _blk((BLK, 8))),
        out_shape=(jax.ShapeDtypeStruct((N, D), jnp.float32),
                   jax.ShapeDtypeStruct((NBLK, 2, D), jnp.float32),
                   jax.ShapeDtypeStruct((N, 8), jnp.float32)),
    )(a0, a1, xr, wl, bl, wr)


def _sage_dense2_body(a0, a1, invc, xr, wl, bl, wr, hpre_out, stats_out):
    su = a0[...] + a1[...]
    mean = su * invc[:, 0:1]
    h = (jnp.dot(mean, wl[...], preferred_element_type=jnp.float32) + bl[...]
         + jnp.dot(xr[...], wr[...], preferred_element_type=jnp.float32))
    hpre_out[...] = h
    stats_out[0, 0, :] = jnp.sum(h, axis=0)
    stats_out[0, 1, :] = jnp.sum(h * h, axis=0)


def _sage_dense2(a0, a1, invc, xr, wl, bl, wr):
    return pl.pallas_call(
        _sage_dense2_body,
        grid=(NBLK,),
        in_specs=[_blk((BLK, D)), _blk((BLK, D)), _blk((BLK, 8)),
                  _blk((BLK, D)), _full((D, D)), _full((1, D)),
                  _full((D, D))],
        out_specs=(_blk((BLK, D)),
                   pl.BlockSpec((1, 2, D), lambda i: (i, 0, 0))),
        out_shape=(jax.ShapeDtypeStruct((N, D), jnp.float32),
                   jax.ShapeDtypeStruct((NBLK, 2, D), jnp.float32)),
    )(a0, a1, invc, xr, wl, bl, wr)


def _bn_finish_body(hpre, stats, resid, g, be, h_out):
    mu = jnp.sum(stats[:, 0, :], axis=0, keepdims=True) * (1.0 / N)
    ex2 = jnp.sum(stats[:, 1, :], axis=0, keepdims=True) * (1.0 / N)
    var = ex2 - mu * mu
    h = g[...] * (hpre[...] - mu) * jax.lax.rsqrt(var + 1e-5) + be[...]
    h_out[...] = jnp.maximum(h + resid[...], 0.0)


def _bn_finish(hpre, stats, resid, g, be):
    return pl.pallas_call(
        _bn_finish_body,
        grid=(NBLK,),
        in_specs=[_blk((BLK, D)), _full((NBLK, 2, D)), _blk((BLK, D)),
                  _full((1, D)), _full((1, D))],
        out_specs=_blk((BLK, D)),
        out_shape=jax.ShapeDtypeStruct((N, D), jnp.float32),
    )(hpre, stats, resid, g, be)


def _bn_gat_body(hpre, stats, resid, g, be, wg, as16, ad16,
                 hgs_out, td_out, cs_out):
    mu = jnp.sum(stats[:, 0, :], axis=0, keepdims=True) * (1.0 / N)
    ex2 = jnp.sum(stats[:, 1, :], axis=0, keepdims=True) * (1.0 / N)
    var = ex2 - mu * mu
    h = g[...] * (hpre[...] - mu) * jax.lax.rsqrt(var + 1e-5) + be[...]
    h2 = jnp.maximum(h + resid[...], 0.0)
    hg = jnp.dot(h2, wg[...], preferred_element_type=jnp.float32)
    a_s = jnp.dot(hg, as16[...], preferred_element_type=jnp.float32)
    a_d = jnp.dot(hg, ad16[...], preferred_element_type=jnp.float32)
    hgs_out[:, :D] = hg
    hgs_out[:, D:] = a_s
    td_out[...] = a_d
    cs_out[0, 0, :] = jnp.max(a_s, axis=0)
    cs_out[0, 1, :] = jnp.max(a_d, axis=0)


def _bn_gat(hpre, stats, resid, g, be, wg, as16, ad16):
    return pl.pallas_call(
        _bn_gat_body,
        grid=(NBLK,),
        in_specs=[_blk((BLK, D)), _full((NBLK, 2, D)), _blk((BLK, D)),
                  _full((1, D)), _full((1, D)), _full((D, D)),
                  _full((D, 16)), _full((D, 16))],
        out_specs=(_blk((BLK, DW)), _blk((BLK, 16)),
                   pl.BlockSpec((1, 2, 16), lambda i: (i, 0, 0))),
        out_shape=(jax.ShapeDtypeStruct((N, DW), jnp.float32),
                   jax.ShapeDtypeStruct((N, 16), jnp.float32),
                   jax.ShapeDtypeStruct((NBLK, 2, 16), jnp.float32)),
    )(hpre, stats, resid, g, be, wg, as16, ad16)


def _head_body(nd0, nd1, hgs, td, cs, bg, wc1, bc1, wc2, bc2,
               emb_out, log_out):
    ms = jnp.max(cs[:, 0, :], axis=0, keepdims=True)
    md = jnp.max(cs[:, 1, :], axis=0, keepdims=True)
    c16 = jnp.maximum(ms + md, 0.0)
    asum = hgs[:, D:] + td[...]
    wself = jnp.exp(jnp.maximum(asum, 0.2 * asum) - c16)
    den = nd0[:, D:] + nd1[:, D:] + wself
    emb = jnp.zeros((BLK, D_OUT), jnp.float32)
    for head in range(HEADS):
        sl = slice(head * D_OUT, (head + 1) * D_OUT)
        blk = (nd0[:, sl] + nd1[:, sl] + hgs[:, sl] * wself[:, head:head + 1])
        emb = emb + blk / (den[:, head:head + 1] + 1e-16)
    emb = emb * (1.0 / HEADS) + bg[...]
    z = jnp.maximum(
        jnp.dot(emb, wc1[...], preferred_element_type=jnp.float32) + bc1[...],
        0.0)
    log_out[...] = (jnp.dot(z, wc2[...], preferred_element_type=jnp.float32)
                    + bc2[...])
    emb_out[...] = emb


def _head(nd0, nd1, hgs, td, cs, bg, wc1, bc1, wc2, bc2):
    return pl.pallas_call(
        _head_body,
        grid=(NBLK,),
        in_specs=[_blk((BLK, DW)), _blk((BLK, DW)), _blk((BLK, DW)),
                  _blk((BLK, 16)), _full((NBLK, 2, 16)), _full((1, D_OUT)),
                  _full((D_OUT, 64)), _full((1, 64)), _full((64, 1)),
                  _full((1, 1))],
        out_specs=(_blk((BLK, D_OUT)), _blk((BLK, 1))),
        out_shape=(jax.ShapeDtypeStruct((N, D_OUT), jnp.float32),
                   jax.ShapeDtypeStruct((N, 1), jnp.float32)),
    )(nd0, nd1, hgs, td, cs, bg, wc1, bc1, wc2, bc2)


def _pad_rows(a):
    return jnp.concatenate(
        [a, jnp.zeros((NPAD - N, a.shape[1]), a.dtype)], axis=0)


def kernel(x, edge_index, W_l1, b_l1, W_r1, g1, be1, W_l2, b_l2, W_r2, g2,
           be2, W_g, att_s, att_d, b_g, Wc1, bc1, Wc2, bc2):
    pad = N + (jnp.arange(EPAD - E, dtype=jnp.int32) % 16)
    src = jnp.concatenate([edge_index[0], pad]).reshape(NTILES * NCH, CHUNK)
    dst = jnp.concatenate([edge_index[1], pad]).reshape(NTILES * NCH, CHUNK)
    idx = jnp.stack([src, dst], axis=1)

    xa = jnp.zeros((NPAD, DW), jnp.float32)
    xa = xa.at[:N, :D].set(x).at[:N, D].set(1.0)

    rows = jnp.arange(D)
    cols = jnp.repeat(jnp.arange(HEADS), D_OUT)
    as16 = jnp.zeros((D, 16), jnp.float32).at[rows, cols].set(att_s.reshape(-1))
    ad16 = jnp.zeros((D, 16), jnp.float32).at[rows, cols].set(att_d.reshape(-1))

    z144 = jnp.zeros((NPAD, DW), jnp.float32)
    z128 = jnp.zeros((NPAD, D), jnp.float32)

    b_l1r = b_l1.reshape(1, D)
    g1r = g1.reshape(1, D)
    be1r = be1.reshape(1, D)
    b_l2r = b_l2.reshape(1, D)
    g2r = g2.reshape(1, D)
    be2r = be2.reshape(1, D)
    bgr = b_g.reshape(1, D_OUT)
    bc1r = bc1.reshape(1, 64)
    bc2r = bc2.reshape(1, 1)

    acc1 = _make_sage_sc(DW)(xa, idx, z144)
    hpre1, st1, invc = _sage_dense(acc1[0], acc1[1], x, W_l1, b_l1r, W_r1)
    h = _bn_finish(hpre1, st1, x, g1r, be1r)

    acc2 = _make_sage_sc(D)(_pad_rows(h), idx, z128)
    hpre2, st2 = _sage_dense2(acc2[0], acc2[1], invc, h, W_l2, b_l2r, W_r2)
    hgs, td, cs = _bn_gat(hpre2, st2, h, g2r, be2r, W_g, as16, ad16)

    nd = _make_gat_sc()(_pad_rows(hgs), _pad_rows(td),
                        cs.reshape(2 * NBLK, 16), idx, z144)
    emb, logits = _head(nd[0], nd[1], hgs, td, cs, bgr, Wc1, bc1r, Wc2, bc2r)
    return (emb, logits)


# speedup vs baseline: 43.5762x; 1.0375x over previous
import functools

import jax
import jax.numpy as jnp
from jax import lax
from jax.experimental import pallas as pl
from jax.experimental.pallas import tpu as pltpu
from jax.experimental.pallas import tpu_sc as plsc

N = 10000
E = 320000
D = 128
DW = D + 16
HEADS = 4
D_OUT = 32

NCORES = 2
NSUB = 16
NTILES = NCORES * NSUB
CHUNK = 128
NCH = 80
NPAIR = NCH // 2
EPAD = NTILES * NCH * CHUNK
NPAD = N + 16
RPT = NPAD // NSUB

BLK = 2000
NBLK = N // BLK

_SC_PARAMS = pltpu.CompilerParams(use_tc_tiling_on_sc=False,
                                  needs_layout_passes=False)


@functools.lru_cache(maxsize=None)
def _mesh():
    return plsc.VectorSubcoreMesh(
        core_axis_name="c", subcore_axis_name="s",
        num_cores=NCORES, num_subcores=NSUB)


def _wait(src, dst, sem):
    pltpu.make_async_copy(src, dst, sem).wait()


@functools.lru_cache(maxsize=None)
def _make_sage_sc(width):
    @functools.partial(
        pl.kernel,
        out_type=jax.ShapeDtypeStruct((NCORES, NPAD, width), jnp.float32),
        mesh=_mesh(),
        scratch_types=[
            pltpu.VMEM_SHARED((NPAD, width), jnp.float32),
            pltpu.VMEM((2, CHUNK), jnp.int32),
            pltpu.VMEM((2, CHUNK), jnp.int32),
            pltpu.VMEM((CHUNK, width), jnp.float32),
            pltpu.VMEM((CHUNK, width), jnp.float32),
            pltpu.SemaphoreType.DMA,
            pltpu.SemaphoreType.DMA,
            pltpu.SemaphoreType.DMA,
            pltpu.SemaphoreType.DMA,
        ],
        compiler_params=_SC_PARAMS,
    )
    def sage_sc(tab_hbm, idx_hbm, zero_hbm, out_hbm,
                acc, ixa, ixb, bufa, bufb, sga, sgb, ssa, ssb):
        c = lax.axis_index("c")
        s = lax.axis_index("s")
        base = (c * NSUB + s) * NCH
        r0 = s * RPT
        pltpu.sync_copy(zero_hbm.at[pl.ds(r0, RPT)], acc.at[pl.ds(r0, RPT)])
        plsc.subcore_barrier()

        pltpu.sync_copy(idx_hbm.at[base], ixa)
        pltpu.sync_copy(idx_hbm.at[base + 1], ixb)
        pltpu.async_copy(tab_hbm.at[ixa.at[0]], bufa, sga)
        pltpu.async_copy(tab_hbm.at[ixb.at[0]], bufb, sgb)

        @pl.loop(0, NPAIR - 1)
        def _(k):
            _wait(tab_hbm.at[ixa.at[0]], bufa, sga)
            pltpu.async_copy(bufa, acc.at[ixa.at[1]], ssa, add=True)
            _wait(tab_hbm.at[ixb.at[0]], bufb, sgb)
            pltpu.async_copy(bufb, acc.at[ixb.at[1]], ssb, add=True)
            _wait(bufa, acc.at[ixa.at[1]], ssa)
            pltpu.sync_copy(idx_hbm.at[base + 2 * k + 2], ixa)
            pltpu.async_copy(tab_hbm.at[ixa.at[0]], bufa, sga)
            _wait(bufb, acc.at[ixb.at[1]], ssb)
            pltpu.sync_copy(idx_hbm.at[base + 2 * k + 3], ixb)
            pltpu.async_copy(tab_hbm.at[ixb.at[0]], bufb, sgb)

        _wait(tab_hbm.at[ixa.at[0]], bufa, sga)
        pltpu.async_copy(bufa, acc.at[ixa.at[1]], ssa, add=True)
        _wait(tab_hbm.at[ixb.at[0]], bufb, sgb)
        pltpu.async_copy(bufb, acc.at[ixb.at[1]], ssb, add=True)
        _wait(bufa, acc.at[ixa.at[1]], ssa)
        _wait(bufb, acc.at[ixb.at[1]], ssb)

        plsc.subcore_barrier()
        pltpu.sync_copy(acc.at[pl.ds(r0, RPT)], out_hbm.at[c, pl.ds(r0, RPT)])

    return sage_sc


@functools.lru_cache(maxsize=None)
def _make_gat_sc():
  @functools.partial(
    pl.kernel,
    out_type=jax.ShapeDtypeStruct((NCORES, NPAD, DW), jnp.float32),
    mesh=_mesh(),
    scratch_types=[
        pltpu.VMEM_SHARED((NPAD, DW), jnp.float32),
        pltpu.VMEM((2, CHUNK), jnp.int32),
        pltpu.VMEM((2, CHUNK), jnp.int32),
        pltpu.VMEM((CHUNK, DW), jnp.float32),
        pltpu.VMEM((CHUNK, DW), jnp.float32),
        pltpu.VMEM((CHUNK, 16), jnp.float32),
        pltpu.VMEM((2 * NBLK, 16), jnp.float32),
        pltpu.SemaphoreType.DMA,
        pltpu.SemaphoreType.DMA,
        pltpu.SemaphoreType.DMA,
        pltpu.SemaphoreType.DMA,
        pltpu.SemaphoreType.DMA,
    ],
    compiler_params=_SC_PARAMS,
  )
  def _gat_sc(hgs_hbm, tabd_hbm, cs_hbm, idx_hbm, zero_hbm, out_hbm,
              acc, ixa, ixb, ha, hb, dbuf, cscr, sga, sgb, sda, ssa, ssb):
    c = lax.axis_index("c")
    s = lax.axis_index("s")
    base = (c * NSUB + s) * NCH
    r0 = s * RPT
    pltpu.sync_copy(zero_hbm.at[pl.ds(r0, RPT)], acc.at[pl.ds(r0, RPT)])
    pltpu.sync_copy(cs_hbm, cscr)
    plsc.subcore_barrier()
    ms = cscr[0, :]
    md = cscr[1, :]
    for b in range(1, NBLK):
        ms = jnp.maximum(ms, cscr[2 * b, :])
        md = jnp.maximum(md, cscr[2 * b + 1, :])
    cvec = jnp.maximum(ms + md, 0.0)

    def compute(hbuf):
        @pl.loop(0, CHUNK, step=4)
        def _(e0):
            for u in range(4):
                e = e0 + u
                a = hbuf[e, pl.ds(D, 16)] + dbuf[e, :]
                hbuf[e, pl.ds(D, 16)] = jnp.exp(
                    jnp.maximum(a, 0.2 * a) - cvec)

        @pl.loop(0, CHUNK, step=4)
        def _(i0):
            for u in range(4):
                i = i0 + u
                ivec = jnp.full((16,), i, jnp.int32)
                for head in range(HEADS):
                    wb = plsc.load_gather(
                        hbuf, [ivec, jnp.full((16,), D + head, jnp.int32)])
                    for half in range(2):
                        col = (head * 2 + half) * 16
                        hbuf[i, pl.ds(col, 16)] = (
                            hbuf[i, pl.ds(col, 16)] * wb)

    pltpu.sync_copy(idx_hbm.at[base], ixa)
    pltpu.sync_copy(idx_hbm.at[base + 1], ixb)
    pltpu.async_copy(hgs_hbm.at[ixa.at[0]], ha, sga)
    pltpu.async_copy(hgs_hbm.at[ixb.at[0]], hb, sgb)
    pltpu.async_copy(tabd_hbm.at[ixa.at[1]], dbuf, sda)

    @pl.loop(0, NPAIR - 1)
    def _(k):
        _wait(hgs_hbm.at[ixa.at[0]], ha, sga)
        _wait(tabd_hbm.at[ixa.at[1]], dbuf, sda)
        compute(ha)
        pltpu.async_copy(ha, acc.at[ixa.at[1]], ssa, add=True)
        pltpu.async_copy(tabd_hbm.at[ixb.at[1]], dbuf, sda)
        _wait(hgs_hbm.at[ixb.at[0]], hb, sgb)
        _wait(tabd_hbm.at[ixb.at[1]], dbuf, sda)
        compute(hb)
        pltpu.async_copy(hb, acc.at[ixb.at[1]], ssb, add=True)
        _wait(ha, acc.at[ixa.at[1]], ssa)
        pltpu.sync_copy(idx_hbm.at[base + 2 * k + 2], ixa)
        pltpu.async_copy(hgs_hbm.at[ixa.at[0]], ha, sga)
        pltpu.async_copy(tabd_hbm.at[ixa.at[1]], dbuf, sda)
        _wait(hb, acc.at[ixb.at[1]], ssb)
        pltpu.sync_copy(idx_hbm.at[base + 2 * k + 3], ixb)
        pltpu.async_copy(hgs_hbm.at[ixb.at[0]], hb, sgb)

    _wait(hgs_hbm.at[ixa.at[0]], ha, sga)
    _wait(tabd_hbm.at[ixa.at[1]], dbuf, sda)
    compute(ha)
    pltpu.async_copy(ha, acc.at[ixa.at[1]], ssa, add=True)
    pltpu.async_copy(tabd_hbm.at[ixb.at[1]], dbuf, sda)
    _wait(hgs_hbm.at[ixb.at[0]], hb, sgb)
    _wait(tabd_hbm.at[ixb.at[1]], dbuf, sda)
    compute(hb)
    pltpu.async_copy(hb, acc.at[ixb.at[1]], ssb, add=True)
    _wait(ha, acc.at[ixa.at[1]], ssa)
    _wait(hb, acc.at[ixb.at[1]], ssb)

    plsc.subcore_barrier()
    pltpu.sync_copy(acc.at[pl.ds(r0, RPT)], out_hbm.at[c, pl.ds(r0, RPT)])

  return _gat_sc


def _blk(shape):
    nd = len(shape)
    return pl.BlockSpec((BLK,) + shape[1:], lambda i: (i,) + (0,) * (nd - 1))


def _full(shape):
    nd = len(shape)
    return pl.BlockSpec(shape, lambda i: (0,) * nd)


def _sage_dense_body(a0, a1, xr, wl, bl, wr, hpre_out, stats_out, invc_out):
    su = a0[:, :D] + a1[:, :D]
    cnt = a0[:, D:D + 1] + a1[:, D:D + 1]
    invc = 1.0 / jnp.maximum(cnt, 1.0)
    mean = su * invc
    h = (jnp.dot(mean, wl[...], preferred_element_type=jnp.float32) + bl[...]
         + jnp.dot(xr[...], wr[...], preferred_element_type=jnp.float32))
    hpre_out[...] = h
    stats_out[0, 0, :] = jnp.sum(h, axis=0)
    stats_out[0, 1, :] = jnp.sum(h * h, axis=0)
    invc_out[...] = jnp.broadcast_to(invc, (BLK, 8))


def _sage_dense(a0, a1, xr, wl, bl, wr):
    return pl.pallas_call(
        _sage_dense_body,
        grid=(NBLK,),
        in_specs=[_blk((BLK, DW)), _blk((BLK, DW)), _blk((BLK, D)),
                  _full((D, D)), _full((1, D)), _full((D, D))],
        out_specs=(_blk((BLK, D)),
                   pl.BlockSpec((1, 2, D), lambda i: (i, 0, 0)),
                   _blk((BLK, 8))),
        out_shape=(jax.ShapeDtypeStruct((N, D), jnp.float32),
                   jax.ShapeDtypeStruct((NBLK, 2, D), jnp.float32),
                   jax.ShapeDtypeStruct((N, 8), jnp.float32)),
    )(a0, a1, xr, wl, bl, wr)


def _sage_dense2_body(a0, a1, invc, xr, wl, bl, wr, hpre_out, stats_out):
    su = a0[...] + a1[...]
    mean = su * invc[:, 0:1]
    h = (jnp.dot(mean, wl[...], preferred_element_type=jnp.float32) + bl[...]
         + jnp.dot(xr[...], wr[...], preferred_element_type=jnp.float32))
    hpre_out[...] = h
    stats_out[0, 0, :] = jnp.sum(h, axis=0)
    stats_out[0, 1, :] = jnp.sum(h * h, axis=0)


def _sage_dense2(a0, a1, invc, xr, wl, bl, wr):
    return pl.pallas_call(
        _sage_dense2_body,
        grid=(NBLK,),
        in_specs=[_blk((BLK, D)), _blk((BLK, D)), _blk((BLK, 8)),
                  _blk((BLK, D)), _full((D, D)), _full((1, D)),
                  _full((D, D))],
        out_specs=(_blk((BLK, D)),
                   pl.BlockSpec((1, 2, D), lambda i: (i, 0, 0))),
        out_shape=(jax.ShapeDtypeStruct((N, D), jnp.float32),
                   jax.ShapeDtypeStruct((NBLK, 2, D), jnp.float32)),
    )(a0, a1, invc, xr, wl, bl, wr)


def _bn_finish_body(hpre, stats, resid, g, be, h_out):
    mu = jnp.sum(stats[:, 0, :], axis=0, keepdims=True) * (1.0 / N)
    ex2 = jnp.sum(stats[:, 1, :], axis=0, keepdims=True) * (1.0 / N)
    var = ex2 - mu * mu
    h = g[...] * (hpre[...] - mu) * jax.lax.rsqrt(var + 1e-5) + be[...]
    h_out[...] = jnp.maximum(h + resid[...], 0.0)


def _bn_finish(hpre, stats, resid, g, be):
    return pl.pallas_call(
        _bn_finish_body,
        grid=(NBLK,),
        in_specs=[_blk((BLK, D)), _full((NBLK, 2, D)), _blk((BLK, D)),
                  _full((1, D)), _full((1, D))],
        out_specs=_blk((BLK, D)),
        out_shape=jax.ShapeDtypeStruct((NPAD, D), jnp.float32),
    )(hpre, stats, resid, g, be)


def _xa_body(xr, xa_out):
    xa_out[:, :D] = xr[...]
    xa_out[:, D:D + 1] = jnp.ones((BLK, 1), jnp.float32)
    xa_out[:, D + 1:] = jnp.zeros((BLK, 15), jnp.float32)


def _xa(xr):
    return pl.pallas_call(
        _xa_body,
        grid=(NBLK,),
        in_specs=[_blk((BLK, D))],
        out_specs=_blk((BLK, DW)),
        out_shape=jax.ShapeDtypeStruct((NPAD, DW), jnp.float32),
    )(xr)


def _bn_gat_body(hpre, stats, resid, g, be, wg, as16, ad16,
                 hgs_out, td_out, cs_out):
    mu = jnp.sum(stats[:, 0, :], axis=0, keepdims=True) * (1.0 / N)
    ex2 = jnp.sum(stats[:, 1, :], axis=0, keepdims=True) * (1.0 / N)
    var = ex2 - mu * mu
    h = g[...] * (hpre[...] - mu) * jax.lax.rsqrt(var + 1e-5) + be[...]
    h2 = jnp.maximum(h + resid[...], 0.0)
    hg = jnp.dot(h2, wg[...], preferred_element_type=jnp.float32)
    a_s = jnp.dot(hg, as16[...], preferred_element_type=jnp.float32)
    a_d = jnp.dot(hg, ad16[...], preferred_element_type=jnp.float32)
    hgs_out[:, :D] = hg
    hgs_out[:, D:] = a_s
    td_out[...] = a_d
    cs_out[0, 0, :] = jnp.max(a_s, axis=0)
    cs_out[0, 1, :] = jnp.max(a_d, axis=0)


def _bn_gat(hpre, stats, resid, g, be, wg, as16, ad16):
    return pl.pallas_call(
        _bn_gat_body,
        grid=(NBLK,),
        in_specs=[_blk((BLK, D)), _full((NBLK, 2, D)), _blk((BLK, D)),
                  _full((1, D)), _full((1, D)), _full((D, D)),
                  _full((D, 16)), _full((D, 16))],
        out_specs=(_blk((BLK, DW)), _blk((BLK, 16)),
                   pl.BlockSpec((1, 2, 16), lambda i: (i, 0, 0))),
        out_shape=(jax.ShapeDtypeStruct((NPAD, DW), jnp.float32),
                   jax.ShapeDtypeStruct((NPAD, 16), jnp.float32),
                   jax.ShapeDtypeStruct((NBLK, 2, 16), jnp.float32)),
    )(hpre, stats, resid, g, be, wg, as16, ad16)


def _head_body(nd0, nd1, hgs, td, cs, bg, wc1, bc1, wc2, bc2,
               emb_out, log_out):
    ms = jnp.max(cs[:, 0, :], axis=0, keepdims=True)
    md = jnp.max(cs[:, 1, :], axis=0, keepdims=True)
    c16 = jnp.maximum(ms + md, 0.0)
    asum = hgs[:, D:] + td[...]
    wself = jnp.exp(jnp.maximum(asum, 0.2 * asum) - c16)
    den = nd0[:, D:] + nd1[:, D:] + wself
    emb = jnp.zeros((BLK, D_OUT), jnp.float32)
    for head in range(HEADS):
        sl = slice(head * D_OUT, (head + 1) * D_OUT)
        blk = (nd0[:, sl] + nd1[:, sl] + hgs[:, sl] * wself[:, head:head + 1])
        emb = emb + blk / (den[:, head:head + 1] + 1e-16)
    emb = emb * (1.0 / HEADS) + bg[...]
    z = jnp.maximum(
        jnp.dot(emb, wc1[...], preferred_element_type=jnp.float32) + bc1[...],
        0.0)
    log_out[...] = (jnp.dot(z, wc2[...], preferred_element_type=jnp.float32)
                    + bc2[...])
    emb_out[...] = emb


def _head(nd0, nd1, hgs, td, cs, bg, wc1, bc1, wc2, bc2):
    return pl.pallas_call(
        _head_body,
        grid=(NBLK,),
        in_specs=[_blk((BLK, DW)), _blk((BLK, DW)), _blk((BLK, DW)),
                  _blk((BLK, 16)), _full((NBLK, 2, 16)), _full((1, D_OUT)),
                  _full((D_OUT, 64)), _full((1, 64)), _full((64, 1)),
                  _full((1, 1))],
        out_specs=(_blk((BLK, D_OUT)), _blk((BLK, 1))),
        out_shape=(jax.ShapeDtypeStruct((N, D_OUT), jnp.float32),
                   jax.ShapeDtypeStruct((N, 1), jnp.float32)),
    )(nd0, nd1, hgs, td, cs, bg, wc1, bc1, wc2, bc2)


def kernel(x, edge_index, W_l1, b_l1, W_r1, g1, be1, W_l2, b_l2, W_r2, g2,
           be2, W_g, att_s, att_d, b_g, Wc1, bc1, Wc2, bc2):
    pad = N + (jnp.arange(EPAD - E, dtype=jnp.int32) % 16)
    src = jnp.concatenate([edge_index[0], pad]).reshape(NTILES * NCH, CHUNK)
    dst = jnp.concatenate([edge_index[1], pad]).reshape(NTILES * NCH, CHUNK)
    idx = jnp.stack([src, dst], axis=1)

    rows = jnp.arange(D)
    cols = jnp.repeat(jnp.arange(HEADS), D_OUT)
    as16 = jnp.zeros((D, 16), jnp.float32).at[rows, cols].set(att_s.reshape(-1))
    ad16 = jnp.zeros((D, 16), jnp.float32).at[rows, cols].set(att_d.reshape(-1))

    z144 = jnp.zeros((NPAD, DW), jnp.float32)
    z128 = jnp.zeros((NPAD, D), jnp.float32)

    b_l1r = b_l1.reshape(1, D)
    g1r = g1.reshape(1, D)
    be1r = be1.reshape(1, D)
    b_l2r = b_l2.reshape(1, D)
    g2r = g2.reshape(1, D)
    be2r = be2.reshape(1, D)
    bgr = b_g.reshape(1, D_OUT)
    bc1r = bc1.reshape(1, 64)
    bc2r = bc2.reshape(1, 1)

    acc1 = _make_sage_sc(DW)(_xa(x), idx, z144)
    hpre1, st1, invc = _sage_dense(acc1[0], acc1[1], x, W_l1, b_l1r, W_r1)
    h = _bn_finish(hpre1, st1, x, g1r, be1r)

    acc2 = _make_sage_sc(D)(h, idx, z128)
    hpre2, st2 = _sage_dense2(acc2[0], acc2[1], invc, h, W_l2, b_l2r, W_r2)
    hgs, td, cs = _bn_gat(hpre2, st2, h, g2r, be2r, W_g, as16, ad16)

    nd = _make_gat_sc()(hgs, td, cs.reshape(2 * NBLK, 16), idx, z144)
    emb, logits = _head(nd[0], nd[1], hgs, td, cs, bgr, Wc1, bc1r, Wc2, bc2r)
    return (emb, logits)


# speedup vs baseline: 52.7116x; 1.2096x over previous
import functools

import jax
import jax.numpy as jnp
from jax import lax
from jax.experimental import pallas as pl
from jax.experimental.pallas import tpu as pltpu
from jax.experimental.pallas import tpu_sc as plsc

N = 10000
E = 320000
D = 128
DW = D + 16
HEADS = 4
D_OUT = 32

NCORES = 2
NSUB = 16
NTILES = NCORES * NSUB
CHUNK = 128
NCH = 80
NPAIR = NCH // 2
EPAD = NTILES * NCH * CHUNK
NPAD = N + 16
RPT = NPAD // NSUB

BLK = 2000
NBLK = N // BLK

_SC_PARAMS = pltpu.CompilerParams(use_tc_tiling_on_sc=False,
                                  needs_layout_passes=False)


@functools.lru_cache(maxsize=None)
def _mesh():
    return plsc.VectorSubcoreMesh(
        core_axis_name="c", subcore_axis_name="s",
        num_cores=NCORES, num_subcores=NSUB)


def _wait(src, dst, sem):
    pltpu.make_async_copy(src, dst, sem).wait()


@functools.lru_cache(maxsize=None)
def _make_sage_sc(width):
    @functools.partial(
        pl.kernel,
        out_type=jax.ShapeDtypeStruct((NCORES, NPAD, width), jnp.float32),
        mesh=_mesh(),
        scratch_types=[
            pltpu.VMEM_SHARED((NPAD, width), jnp.float32),
            pltpu.VMEM((2, CHUNK), jnp.int32),
            pltpu.VMEM((2, CHUNK), jnp.int32),
            pltpu.VMEM((CHUNK, width), jnp.float32),
            pltpu.VMEM((CHUNK, width), jnp.float32),
            pltpu.SemaphoreType.DMA,
            pltpu.SemaphoreType.DMA,
            pltpu.SemaphoreType.DMA,
            pltpu.SemaphoreType.DMA,
        ],
        compiler_params=_SC_PARAMS,
    )
    def sage_sc(tab_hbm, idx_hbm, zero_hbm, out_hbm,
                acc, ixa, ixb, bufa, bufb, sga, sgb, ssa, ssb):
        c = lax.axis_index("c")
        s = lax.axis_index("s")
        base = (c * NSUB + s) * NCH
        r0 = s * RPT
        pltpu.sync_copy(zero_hbm.at[pl.ds(r0, RPT)], acc.at[pl.ds(r0, RPT)])
        plsc.subcore_barrier()

        pltpu.sync_copy(idx_hbm.at[base], ixa)
        pltpu.sync_copy(idx_hbm.at[base + 1], ixb)
        pltpu.async_copy(tab_hbm.at[ixa.at[0]], bufa, sga)
        pltpu.async_copy(tab_hbm.at[ixb.at[0]], bufb, sgb)

        @pl.loop(0, NPAIR - 1)
        def _(k):
            _wait(tab_hbm.at[ixa.at[0]], bufa, sga)
            pltpu.async_copy(bufa, acc.at[ixa.at[1]], ssa, add=True)
            _wait(tab_hbm.at[ixb.at[0]], bufb, sgb)
            pltpu.async_copy(bufb, acc.at[ixb.at[1]], ssb, add=True)
            _wait(bufa, acc.at[ixa.at[1]], ssa)
            pltpu.sync_copy(idx_hbm.at[base + 2 * k + 2], ixa)
            pltpu.async_copy(tab_hbm.at[ixa.at[0]], bufa, sga)
            _wait(bufb, acc.at[ixb.at[1]], ssb)
            pltpu.sync_copy(idx_hbm.at[base + 2 * k + 3], ixb)
            pltpu.async_copy(tab_hbm.at[ixb.at[0]], bufb, sgb)

        _wait(tab_hbm.at[ixa.at[0]], bufa, sga)
        pltpu.async_copy(bufa, acc.at[ixa.at[1]], ssa, add=True)
        _wait(tab_hbm.at[ixb.at[0]], bufb, sgb)
        pltpu.async_copy(bufb, acc.at[ixb.at[1]], ssb, add=True)
        _wait(bufa, acc.at[ixa.at[1]], ssa)
        _wait(bufb, acc.at[ixb.at[1]], ssb)

        plsc.subcore_barrier()
        pltpu.sync_copy(acc.at[pl.ds(r0, RPT)], out_hbm.at[c, pl.ds(r0, RPT)])

    return sage_sc


@functools.lru_cache(maxsize=None)
def _make_gat_sc():
  @functools.partial(
    pl.kernel,
    out_type=jax.ShapeDtypeStruct((NCORES, NPAD, DW), jnp.float32),
    mesh=_mesh(),
    scratch_types=[
        pltpu.VMEM_SHARED((NPAD, DW), jnp.float32),
        pltpu.VMEM((2, CHUNK), jnp.int32),
        pltpu.VMEM((2, CHUNK), jnp.int32),
        pltpu.VMEM((CHUNK, DW), jnp.float32),
        pltpu.VMEM((CHUNK, DW), jnp.float32),
        pltpu.VMEM((CHUNK, 16), jnp.float32),
        pltpu.VMEM((2 * NBLK, 16), jnp.float32),
        pltpu.SemaphoreType.DMA,
        pltpu.SemaphoreType.DMA,
        pltpu.SemaphoreType.DMA,
        pltpu.SemaphoreType.DMA,
        pltpu.SemaphoreType.DMA,
    ],
    compiler_params=_SC_PARAMS,
  )
  def _gat_sc(hgs_hbm, tabd_hbm, cs_hbm, idx_hbm, zero_hbm, out_hbm,
              acc, ixa, ixb, ha, hb, dbuf, cscr, sga, sgb, sda, ssa, ssb):
    c = lax.axis_index("c")
    s = lax.axis_index("s")
    base = (c * NSUB + s) * NCH
    r0 = s * RPT
    pltpu.sync_copy(zero_hbm.at[pl.ds(r0, RPT)], acc.at[pl.ds(r0, RPT)])
    pltpu.sync_copy(cs_hbm, cscr)
    plsc.subcore_barrier()
    ms = cscr[0, :]
    md = cscr[1, :]
    for b in range(1, NBLK):
        ms = jnp.maximum(ms, cscr[2 * b, :])
        md = jnp.maximum(md, cscr[2 * b + 1, :])
    cvec = jnp.maximum(ms + md, 0.0)

    def compute(hbuf):
        @pl.loop(0, CHUNK, step=4)
        def _(e0):
            for u in range(4):
                e = e0 + u
                a = hbuf[e, pl.ds(D, 16)] + dbuf[e, :]
                hbuf[e, pl.ds(D, 16)] = jnp.exp(
                    jnp.maximum(a, 0.2 * a) - cvec)

        @pl.loop(0, CHUNK, step=4)
        def _(i0):
            for u in range(4):
                i = i0 + u
                w16 = hbuf[i, pl.ds(D, 16)]
                for head in range(HEADS):
                    wb = lax.gather(
                        w16, jnp.full((16, 1), head, jnp.int32),
                        lax.GatherDimensionNumbers(
                            offset_dims=(), collapsed_slice_dims=(0,),
                            start_index_map=(0,)),
                        (1,), mode=lax.GatherScatterMode.PROMISE_IN_BOUNDS)
                    for half in range(2):
                        col = (head * 2 + half) * 16
                        hbuf[i, pl.ds(col, 16)] = (
                            hbuf[i, pl.ds(col, 16)] * wb)

    pltpu.sync_copy(idx_hbm.at[base], ixa)
    pltpu.sync_copy(idx_hbm.at[base + 1], ixb)
    pltpu.async_copy(hgs_hbm.at[ixa.at[0]], ha, sga)
    pltpu.async_copy(hgs_hbm.at[ixb.at[0]], hb, sgb)
    pltpu.async_copy(tabd_hbm.at[ixa.at[1]], dbuf, sda)

    @pl.loop(0, NPAIR - 1)
    def _(k):
        _wait(hgs_hbm.at[ixa.at[0]], ha, sga)
        _wait(tabd_hbm.at[ixa.at[1]], dbuf, sda)
        compute(ha)
        pltpu.async_copy(ha, acc.at[ixa.at[1]], ssa, add=True)
        pltpu.async_copy(tabd_hbm.at[ixb.at[1]], dbuf, sda)
        _wait(hgs_hbm.at[ixb.at[0]], hb, sgb)
        _wait(tabd_hbm.at[ixb.at[1]], dbuf, sda)
        compute(hb)
        pltpu.async_copy(hb, acc.at[ixb.at[1]], ssb, add=True)
        _wait(ha, acc.at[ixa.at[1]], ssa)
        pltpu.sync_copy(idx_hbm.at[base + 2 * k + 2], ixa)
        pltpu.async_copy(hgs_hbm.at[ixa.at[0]], ha, sga)
        pltpu.async_copy(tabd_hbm.at[ixa.at[1]], dbuf, sda)
        _wait(hb, acc.at[ixb.at[1]], ssb)
        pltpu.sync_copy(idx_hbm.at[base + 2 * k + 3], ixb)
        pltpu.async_copy(hgs_hbm.at[ixb.at[0]], hb, sgb)

    _wait(hgs_hbm.at[ixa.at[0]], ha, sga)
    _wait(tabd_hbm.at[ixa.at[1]], dbuf, sda)
    compute(ha)
    pltpu.async_copy(ha, acc.at[ixa.at[1]], ssa, add=True)
    pltpu.async_copy(tabd_hbm.at[ixb.at[1]], dbuf, sda)
    _wait(hgs_hbm.at[ixb.at[0]], hb, sgb)
    _wait(tabd_hbm.at[ixb.at[1]], dbuf, sda)
    compute(hb)
    pltpu.async_copy(hb, acc.at[ixb.at[1]], ssb, add=True)
    _wait(ha, acc.at[ixa.at[1]], ssa)
    _wait(hb, acc.at[ixb.at[1]], ssb)

    plsc.subcore_barrier()
    pltpu.sync_copy(acc.at[pl.ds(r0, RPT)], out_hbm.at[c, pl.ds(r0, RPT)])

  return _gat_sc


def _blk(shape):
    nd = len(shape)
    return pl.BlockSpec((BLK,) + shape[1:], lambda i: (i,) + (0,) * (nd - 1))


def _full(shape):
    nd = len(shape)
    return pl.BlockSpec(shape, lambda i: (0,) * nd)


def _sage_dense_body(a0, a1, xr, wl, bl, wr, hpre_out, stats_out, invc_out):
    su = a0[:, :D] + a1[:, :D]
    cnt = a0[:, D:D + 1] + a1[:, D:D + 1]
    invc = 1.0 / jnp.maximum(cnt, 1.0)
    mean = su * invc
    h = (jnp.dot(mean, wl[...], preferred_element_type=jnp.float32) + bl[...]
         + jnp.dot(xr[...], wr[...], preferred_element_type=jnp.float32))
    hpre_out[...] = h
    stats_out[0, 0, :] = jnp.sum(h, axis=0)
    stats_out[0, 1, :] = jnp.sum(h * h, axis=0)
    invc_out[...] = jnp.broadcast_to(invc, (BLK, 8))


def _sage_dense(a0, a1, xr, wl, bl, wr):
    return pl.pallas_call(
        _sage_dense_body,
        grid=(NBLK,),
        in_specs=[_blk((BLK, DW)), _blk((BLK, DW)), _blk((BLK, D)),
                  _full((D, D)), _full((1, D)), _full((D, D))],
        out_specs=(_blk((BLK, D)),
                   pl.BlockSpec((1, 2, D), lambda i: (i, 0, 0)),
                   _blk((BLK, 8))),
        out_shape=(jax.ShapeDtypeStruct((N, D), jnp.float32),
                   jax.ShapeDtypeStruct((NBLK, 2, D), jnp.float32),
                   jax.ShapeDtypeStruct((N, 8), jnp.float32)),
    )(a0, a1, xr, wl, bl, wr)


def _sage_dense2_body(a0, a1, invc, xr, wl, bl, wr, hpre_out, stats_out):
    su = a0[...] + a1[...]
    mean = su * invc[:, 0:1]
    h = (jnp.dot(mean, wl[...], preferred_element_type=jnp.float32) + bl[...]
         + jnp.dot(xr[...], wr[...], preferred_element_type=jnp.float32))
    hpre_out[...] = h
    stats_out[0, 0, :] = jnp.sum(h, axis=0)
    stats_out[0, 1, :] = jnp.sum(h * h, axis=0)


def _sage_dense2(a0, a1, invc, xr, wl, bl, wr):
    return pl.pallas_call(
        _sage_dense2_body,
        grid=(NBLK,),
        in_specs=[_blk((BLK, D)), _blk((BLK, D)), _blk((BLK, 8)),
                  _blk((BLK, D)), _full((D, D)), _full((1, D)),
                  _full((D, D))],
        out_specs=(_blk((BLK, D)),
                   pl.BlockSpec((1, 2, D), lambda i: (i, 0, 0))),
        out_shape=(jax.ShapeDtypeStruct((N, D), jnp.float32),
                   jax.ShapeDtypeStruct((NBLK, 2, D), jnp.float32)),
    )(a0, a1, invc, xr, wl, bl, wr)


def _bn_finish_body(hpre, stats, resid, g, be, h_out):
    mu = jnp.sum(stats[:, 0, :], axis=0, keepdims=True) * (1.0 / N)
    ex2 = jnp.sum(stats[:, 1, :], axis=0, keepdims=True) * (1.0 / N)
    var = ex2 - mu * mu
    h = g[...] * (hpre[...] - mu) * jax.lax.rsqrt(var + 1e-5) + be[...]
    h_out[...] = jnp.maximum(h + resid[...], 0.0)


def _bn_finish(hpre, stats, resid, g, be):
    return pl.pallas_call(
        _bn_finish_body,
        grid=(NBLK,),
        in_specs=[_blk((BLK, D)), _full((NBLK, 2, D)), _blk((BLK, D)),
                  _full((1, D)), _full((1, D))],
        out_specs=_blk((BLK, D)),
        out_shape=jax.ShapeDtypeStruct((NPAD, D), jnp.float32),
    )(hpre, stats, resid, g, be)


def _xa_body(xr, xa_out):
    xa_out[:, :D] = xr[...]
    xa_out[:, D:D + 1] = jnp.ones((BLK, 1), jnp.float32)
    xa_out[:, D + 1:] = jnp.zeros((BLK, 15), jnp.float32)


def _xa(xr):
    return pl.pallas_call(
        _xa_body,
        grid=(NBLK,),
        in_specs=[_blk((BLK, D))],
        out_specs=_blk((BLK, DW)),
        out_shape=jax.ShapeDtypeStruct((NPAD, DW), jnp.float32),
    )(xr)


def _bn_gat_body(hpre, stats, resid, g, be, wg, as16, ad16,
                 hgs_out, td_out, cs_out):
    mu = jnp.sum(stats[:, 0, :], axis=0, keepdims=True) * (1.0 / N)
    ex2 = jnp.sum(stats[:, 1, :], axis=0, keepdims=True) * (1.0 / N)
    var = ex2 - mu * mu
    h = g[...] * (hpre[...] - mu) * jax.lax.rsqrt(var + 1e-5) + be[...]
    h2 = jnp.maximum(h + resid[...], 0.0)
    hg = jnp.dot(h2, wg[...], preferred_element_type=jnp.float32)
    a_s = jnp.dot(hg, as16[...], preferred_element_type=jnp.float32)
    a_d = jnp.dot(hg, ad16[...], preferred_element_type=jnp.float32)
    hgs_out[:, :D] = hg
    hgs_out[:, D:] = a_s
    td_out[...] = a_d
    cs_out[0, 0, :] = jnp.max(a_s, axis=0)
    cs_out[0, 1, :] = jnp.max(a_d, axis=0)


def _bn_gat(hpre, stats, resid, g, be, wg, as16, ad16):
    return pl.pallas_call(
        _bn_gat_body,
        grid=(NBLK,),
        in_specs=[_blk((BLK, D)), _full((NBLK, 2, D)), _blk((BLK, D)),
                  _full((1, D)), _full((1, D)), _full((D, D)),
                  _full((D, 16)), _full((D, 16))],
        out_specs=(_blk((BLK, DW)), _blk((BLK, 16)),
                   pl.BlockSpec((1, 2, 16), lambda i: (i, 0, 0))),
        out_shape=(jax.ShapeDtypeStruct((NPAD, DW), jnp.float32),
                   jax.ShapeDtypeStruct((NPAD, 16), jnp.float32),
                   jax.ShapeDtypeStruct((NBLK, 2, 16), jnp.float32)),
    )(hpre, stats, resid, g, be, wg, as16, ad16)


def _head_body(nd0, nd1, hgs, td, cs, bg, wc1, bc1, wc2, bc2,
               emb_out, log_out):
    ms = jnp.max(cs[:, 0, :], axis=0, keepdims=True)
    md = jnp.max(cs[:, 1, :], axis=0, keepdims=True)
    c16 = jnp.maximum(ms + md, 0.0)
    asum = hgs[:, D:] + td[...]
    wself = jnp.exp(jnp.maximum(asum, 0.2 * asum) - c16)
    den = nd0[:, D:] + nd1[:, D:] + wself
    emb = jnp.zeros((BLK, D_OUT), jnp.float32)
    for head in range(HEADS):
        sl = slice(head * D_OUT, (head + 1) * D_OUT)
        blk = (nd0[:, sl] + nd1[:, sl] + hgs[:, sl] * wself[:, head:head + 1])
        emb = emb + blk / (den[:, head:head + 1] + 1e-16)
    emb = emb * (1.0 / HEADS) + bg[...]
    z = jnp.maximum(
        jnp.dot(emb, wc1[...], preferred_element_type=jnp.float32) + bc1[...],
        0.0)
    log_out[...] = (jnp.dot(z, wc2[...], preferred_element_type=jnp.float32)
                    + bc2[...])
    emb_out[...] = emb


def _head(nd0, nd1, hgs, td, cs, bg, wc1, bc1, wc2, bc2):
    return pl.pallas_call(
        _head_body,
        grid=(NBLK,),
        in_specs=[_blk((BLK, DW)), _blk((BLK, DW)), _blk((BLK, DW)),
                  _blk((BLK, 16)), _full((NBLK, 2, 16)), _full((1, D_OUT)),
                  _full((D_OUT, 64)), _full((1, 64)), _full((64, 1)),
                  _full((1, 1))],
        out_specs=(_blk((BLK, D_OUT)), _blk((BLK, 1))),
        out_shape=(jax.ShapeDtypeStruct((N, D_OUT), jnp.float32),
                   jax.ShapeDtypeStruct((N, 1), jnp.float32)),
    )(nd0, nd1, hgs, td, cs, bg, wc1, bc1, wc2, bc2)


def kernel(x, edge_index, W_l1, b_l1, W_r1, g1, be1, W_l2, b_l2, W_r2, g2,
           be2, W_g, att_s, att_d, b_g, Wc1, bc1, Wc2, bc2):
    pad = N + (jnp.arange(EPAD - E, dtype=jnp.int32) % 16)
    src = jnp.concatenate([edge_index[0], pad]).reshape(NTILES * NCH, CHUNK)
    dst = jnp.concatenate([edge_index[1], pad]).reshape(NTILES * NCH, CHUNK)
    idx = jnp.stack([src, dst], axis=1)

    rows = jnp.arange(D)
    cols = jnp.repeat(jnp.arange(HEADS), D_OUT)
    as16 = jnp.zeros((D, 16), jnp.float32).at[rows, cols].set(att_s.reshape(-1))
    ad16 = jnp.zeros((D, 16), jnp.float32).at[rows, cols].set(att_d.reshape(-1))

    z144 = jnp.zeros((NPAD, DW), jnp.float32)
    z128 = jnp.zeros((NPAD, D), jnp.float32)

    b_l1r = b_l1.reshape(1, D)
    g1r = g1.reshape(1, D)
    be1r = be1.reshape(1, D)
    b_l2r = b_l2.reshape(1, D)
    g2r = g2.reshape(1, D)
    be2r = be2.reshape(1, D)
    bgr = b_g.reshape(1, D_OUT)
    bc1r = bc1.reshape(1, 64)
    bc2r = bc2.reshape(1, 1)

    acc1 = _make_sage_sc(DW)(_xa(x), idx, z144)
    hpre1, st1, invc = _sage_dense(acc1[0], acc1[1], x, W_l1, b_l1r, W_r1)
    h = _bn_finish(hpre1, st1, x, g1r, be1r)

    acc2 = _make_sage_sc(D)(h, idx, z128)
    hpre2, st2 = _sage_dense2(acc2[0], acc2[1], invc, h, W_l2, b_l2r, W_r2)
    hgs, td, cs = _bn_gat(hpre2, st2, h, g2r, be2r, W_g, as16, ad16)

    nd = _make_gat_sc()(hgs, td, cs.reshape(2 * NBLK, 16), idx, z144)
    emb, logits = _head(nd[0], nd[1], hgs, td, cs, bgr, Wc1, bc1r, Wc2, bc2r)
    return (emb, logits)


# speedup vs baseline: 53.9465x; 1.0234x over previous
import functools

import jax
import jax.numpy as jnp
from jax import lax
from jax.experimental import pallas as pl
from jax.experimental.pallas import tpu as pltpu
from jax.experimental.pallas import tpu_sc as plsc

N = 10000
E = 320000
D = 128
DW = D + 16
HEADS = 4
D_OUT = 32

NCORES = 2
NSUB = 16
NTILES = NCORES * NSUB
CHS = 128
NCHS = 80
NPAIR = NCHS // 2
EPAD = NTILES * NCHS * CHS
NPAD = N + 16
RPT = NPAD // NSUB

BLK = 2000
NBLK = N // BLK

CHG = 120
NCHG = 84
NPAIRG = NCHG // 2
EPADG = NTILES * NCHG * CHG

_SC_PARAMS = pltpu.CompilerParams(use_tc_tiling_on_sc=False,
                                  needs_layout_passes=False)


@functools.lru_cache(maxsize=None)
def _mesh():
    return plsc.VectorSubcoreMesh(
        core_axis_name="c", subcore_axis_name="s",
        num_cores=NCORES, num_subcores=NSUB)


def _wait(src, dst, sem):
    pltpu.make_async_copy(src, dst, sem).wait()


@functools.lru_cache(maxsize=None)
def _make_sage_sc(width):
    @functools.partial(
        pl.kernel,
        out_type=jax.ShapeDtypeStruct((NCORES, NPAD, width), jnp.float32),
        mesh=_mesh(),
        scratch_types=[
            pltpu.VMEM_SHARED((NPAD, width), jnp.float32),
            pltpu.VMEM((2, CHS), jnp.int32),
            pltpu.VMEM((2, CHS), jnp.int32),
            pltpu.VMEM((CHS, width), jnp.float32),
            pltpu.VMEM((CHS, width), jnp.float32),
            pltpu.SemaphoreType.DMA,
            pltpu.SemaphoreType.DMA,
            pltpu.SemaphoreType.DMA,
            pltpu.SemaphoreType.DMA,
        ],
        compiler_params=_SC_PARAMS,
    )
    def sage_sc(tab_hbm, idx_hbm, zero_hbm, out_hbm,
                acc, ixa, ixb, bufa, bufb, sga, sgb, ssa, ssb):
        c = lax.axis_index("c")
        s = lax.axis_index("s")
        base = (c * NSUB + s) * NCHS
        r0 = s * RPT
        pltpu.sync_copy(zero_hbm.at[pl.ds(r0, RPT)], acc.at[pl.ds(r0, RPT)])
        plsc.subcore_barrier()

        pltpu.sync_copy(idx_hbm.at[base], ixa)
        pltpu.sync_copy(idx_hbm.at[base + 1], ixb)
        pltpu.async_copy(tab_hbm.at[ixa.at[0]], bufa, sga)
        pltpu.async_copy(tab_hbm.at[ixb.at[0]], bufb, sgb)

        @pl.loop(0, NPAIR - 1)
        def _(k):
            _wait(tab_hbm.at[ixa.at[0]], bufa, sga)
            pltpu.async_copy(bufa, acc.at[ixa.at[1]], ssa, add=True)
            _wait(tab_hbm.at[ixb.at[0]], bufb, sgb)
            pltpu.async_copy(bufb, acc.at[ixb.at[1]], ssb, add=True)
            _wait(bufa, acc.at[ixa.at[1]], ssa)
            pltpu.sync_copy(idx_hbm.at[base + 2 * k + 2], ixa)
            pltpu.async_copy(tab_hbm.at[ixa.at[0]], bufa, sga)
            _wait(bufb, acc.at[ixb.at[1]], ssb)
            pltpu.sync_copy(idx_hbm.at[base + 2 * k + 3], ixb)
            pltpu.async_copy(tab_hbm.at[ixb.at[0]], bufb, sgb)

        _wait(tab_hbm.at[ixa.at[0]], bufa, sga)
        pltpu.async_copy(bufa, acc.at[ixa.at[1]], ssa, add=True)
        _wait(tab_hbm.at[ixb.at[0]], bufb, sgb)
        pltpu.async_copy(bufb, acc.at[ixb.at[1]], ssb, add=True)
        _wait(bufa, acc.at[ixa.at[1]], ssa)
        _wait(bufb, acc.at[ixb.at[1]], ssb)

        plsc.subcore_barrier()
        pltpu.sync_copy(acc.at[pl.ds(r0, RPT)], out_hbm.at[c, pl.ds(r0, RPT)])

    return sage_sc


@functools.lru_cache(maxsize=None)
def _make_gat_sc():
  @functools.partial(
    pl.kernel,
    out_type=jax.ShapeDtypeStruct((NCORES, NPAD, DW), jnp.float32),
    mesh=_mesh(),
    scratch_types=[
        pltpu.VMEM_SHARED((NPAD, DW), jnp.float32),
        pltpu.VMEM((2, CHG), jnp.int32),
        pltpu.VMEM((2, CHG), jnp.int32),
        pltpu.VMEM((CHG, DW), jnp.float32),
        pltpu.VMEM((CHG, DW), jnp.float32),
        pltpu.VMEM((CHG, 16), jnp.float32),
        pltpu.VMEM((CHG, 16), jnp.float32),
        pltpu.VMEM((1, 16), jnp.float32),
        pltpu.SemaphoreType.DMA,
        pltpu.SemaphoreType.DMA,
        pltpu.SemaphoreType.DMA,
        pltpu.SemaphoreType.DMA,
        pltpu.SemaphoreType.DMA,
        pltpu.SemaphoreType.DMA,
    ],
    compiler_params=_SC_PARAMS,
  )
  def _gat_sc(hgs_hbm, tabd_hbm, c16_hbm, idx_hbm, zero_hbm, out_hbm,
              acc, ixa, ixb, ha, hb, da, db, cscr,
              sga, sgb, sda, sdb, ssa, ssb):
    c = lax.axis_index("c")
    s = lax.axis_index("s")
    base = (c * NSUB + s) * NCHG
    r0 = s * RPT
    pltpu.sync_copy(zero_hbm.at[pl.ds(r0, RPT)], acc.at[pl.ds(r0, RPT)])
    pltpu.sync_copy(c16_hbm, cscr)
    plsc.subcore_barrier()
    cvec = cscr[0, :]

    def compute(hbuf, dbuf):
        @pl.loop(0, CHG, step=4)
        def _(i0):
            for u in range(4):
                i = i0 + u
                a = hbuf[i, pl.ds(D, 16)] + dbuf[i, :]
                w16 = jnp.exp(jnp.maximum(a, 0.2 * a) - cvec)
                hbuf[i, pl.ds(D, 16)] = w16
                for head in range(HEADS):
                    wb = lax.gather(
                        w16, jnp.full((16, 1), head, jnp.int32),
                        lax.GatherDimensionNumbers(
                            offset_dims=(), collapsed_slice_dims=(0,),
                            start_index_map=(0,)),
                        (1,), mode=lax.GatherScatterMode.PROMISE_IN_BOUNDS)
                    for half in range(2):
                        col = (head * 2 + half) * 16
                        hbuf[i, pl.ds(col, 16)] = (
                            hbuf[i, pl.ds(col, 16)] * wb)

    pltpu.sync_copy(idx_hbm.at[base], ixa)
    pltpu.sync_copy(idx_hbm.at[base + 1], ixb)
    pltpu.async_copy(hgs_hbm.at[ixa.at[0]], ha, sga)
    pltpu.async_copy(tabd_hbm.at[ixa.at[1]], da, sda)
    pltpu.async_copy(hgs_hbm.at[ixb.at[0]], hb, sgb)
    pltpu.async_copy(tabd_hbm.at[ixb.at[1]], db, sdb)

    @pl.loop(0, NPAIRG - 1)
    def _(k):
        _wait(hgs_hbm.at[ixa.at[0]], ha, sga)
        _wait(tabd_hbm.at[ixa.at[1]], da, sda)
        compute(ha, da)
        pltpu.async_copy(ha, acc.at[ixa.at[1]], ssa, add=True)
        _wait(hgs_hbm.at[ixb.at[0]], hb, sgb)
        _wait(tabd_hbm.at[ixb.at[1]], db, sdb)
        compute(hb, db)
        pltpu.async_copy(hb, acc.at[ixb.at[1]], ssb, add=True)
        _wait(ha, acc.at[ixa.at[1]], ssa)
        pltpu.sync_copy(idx_hbm.at[base + 2 * k + 2], ixa)
        pltpu.async_copy(hgs_hbm.at[ixa.at[0]], ha, sga)
        pltpu.async_copy(tabd_hbm.at[ixa.at[1]], da, sda)
        _wait(hb, acc.at[ixb.at[1]], ssb)
        pltpu.sync_copy(idx_hbm.at[base + 2 * k + 3], ixb)
        pltpu.async_copy(hgs_hbm.at[ixb.at[0]], hb, sgb)
        pltpu.async_copy(tabd_hbm.at[ixb.at[1]], db, sdb)

    _wait(hgs_hbm.at[ixa.at[0]], ha, sga)
    _wait(tabd_hbm.at[ixa.at[1]], da, sda)
    compute(ha, da)
    pltpu.async_copy(ha, acc.at[ixa.at[1]], ssa, add=True)
    _wait(hgs_hbm.at[ixb.at[0]], hb, sgb)
    _wait(tabd_hbm.at[ixb.at[1]], db, sdb)
    compute(hb, db)
    pltpu.async_copy(hb, acc.at[ixb.at[1]], ssb, add=True)
    _wait(ha, acc.at[ixa.at[1]], ssa)
    _wait(hb, acc.at[ixb.at[1]], ssb)

    plsc.subcore_barrier()
    pltpu.sync_copy(acc.at[pl.ds(r0, RPT)], out_hbm.at[c, pl.ds(r0, RPT)])

  return _gat_sc


def _blk(shape):
    nd = len(shape)
    return pl.BlockSpec((BLK,) + shape[1:], lambda i: (i,) + (0,) * (nd - 1))


def _full(shape):
    nd = len(shape)
    return pl.BlockSpec(shape, lambda i: (0,) * nd)


def _sage_dense_body(a0, a1, xr, wl, bl, wr, hpre_out, stats_out, invc_out):
    su = a0[:, :D] + a1[:, :D]
    cnt = a0[:, D:D + 1] + a1[:, D:D + 1]
    invc = 1.0 / jnp.maximum(cnt, 1.0)
    mean = su * invc
    h = (jnp.dot(mean, wl[...], preferred_element_type=jnp.float32) + bl[...]
         + jnp.dot(xr[...], wr[...], preferred_element_type=jnp.float32))
    hpre_out[...] = h
    stats_out[0, 0, :] = jnp.sum(h, axis=0)
    stats_out[0, 1, :] = jnp.sum(h * h, axis=0)
    invc_out[...] = jnp.broadcast_to(invc, (BLK, 8))


def _sage_dense(a0, a1, xr, wl, bl, wr):
    return pl.pallas_call(
        _sage_dense_body,
        grid=(NBLK,),
        in_specs=[_blk((BLK, DW)), _blk((BLK, DW)), _blk((BLK, D)),
                  _full((D, D)), _full((1, D)), _full((D, D))],
        out_specs=(_blk((BLK, D)),
                   pl.BlockSpec((1, 2, D), lambda i: (i, 0, 0)),
                   _blk((BLK, 8))),
        out_shape=(jax.ShapeDtypeStruct((N, D), jnp.float32),
                   jax.ShapeDtypeStruct((NBLK, 2, D), jnp.float32),
                   jax.ShapeDtypeStruct((N, 8), jnp.float32)),
    )(a0, a1, xr, wl, bl, wr)


def _sage_dense2_body(a0, a1, invc, xr, wl, bl, wr, hpre_out, stats_out):
    su = a0[...] + a1[...]
    mean = su * invc[:, 0:1]
    h = (jnp.dot(mean, wl[...], preferred_element_type=jnp.float32) + bl[...]
         + jnp.dot(xr[...], wr[...], preferred_element_type=jnp.float32))
    hpre_out[...] = h
    stats_out[0, 0, :] = jnp.sum(h, axis=0)
    stats_out[0, 1, :] = jnp.sum(h * h, axis=0)


def _sage_dense2(a0, a1, invc, xr, wl, bl, wr):
    return pl.pallas_call(
        _sage_dense2_body,
        grid=(NBLK,),
        in_specs=[_blk((BLK, D)), _blk((BLK, D)), _blk((BLK, 8)),
                  _blk((BLK, D)), _full((D, D)), _full((1, D)),
                  _full((D, D))],
        out_specs=(_blk((BLK, D)),
                   pl.BlockSpec((1, 2, D), lambda i: (i, 0, 0))),
        out_shape=(jax.ShapeDtypeStruct((N, D), jnp.float32),
                   jax.ShapeDtypeStruct((NBLK, 2, D), jnp.float32)),
    )(a0, a1, invc, xr, wl, bl, wr)


def _bn_finish_body(hpre, stats, resid, g, be, h_out):
    mu = jnp.sum(stats[:, 0, :], axis=0, keepdims=True) * (1.0 / N)
    ex2 = jnp.sum(stats[:, 1, :], axis=0, keepdims=True) * (1.0 / N)
    var = ex2 - mu * mu
    h = g[...] * (hpre[...] - mu) * jax.lax.rsqrt(var + 1e-5) + be[...]
    h_out[...] = jnp.maximum(h + resid[...], 0.0)


def _bn_finish(hpre, stats, resid, g, be):
    return pl.pallas_call(
        _bn_finish_body,
        grid=(NBLK,),
        in_specs=[_blk((BLK, D)), _full((NBLK, 2, D)), _blk((BLK, D)),
                  _full((1, D)), _full((1, D))],
        out_specs=_blk((BLK, D)),
        out_shape=jax.ShapeDtypeStruct((NPAD, D), jnp.float32),
    )(hpre, stats, resid, g, be)


def _xa_body(xr, xa_out):
    xa_out[:, :D] = xr[...]
    xa_out[:, D:D + 1] = jnp.ones((BLK, 1), jnp.float32)
    xa_out[:, D + 1:] = jnp.zeros((BLK, 15), jnp.float32)


def _xa(xr):
    return pl.pallas_call(
        _xa_body,
        grid=(NBLK,),
        in_specs=[_blk((BLK, D))],
        out_specs=_blk((BLK, DW)),
        out_shape=jax.ShapeDtypeStruct((NPAD, DW), jnp.float32),
    )(xr)


def _bn_gat_body(hpre, stats, resid, g, be, wg, as16, ad16,
                 hgs_out, td_out, cs_out):
    mu = jnp.sum(stats[:, 0, :], axis=0, keepdims=True) * (1.0 / N)
    ex2 = jnp.sum(stats[:, 1, :], axis=0, keepdims=True) * (1.0 / N)
    var = ex2 - mu * mu
    h = g[...] * (hpre[...] - mu) * jax.lax.rsqrt(var + 1e-5) + be[...]
    h2 = jnp.maximum(h + resid[...], 0.0)
    hg = jnp.dot(h2, wg[...], preferred_element_type=jnp.float32)
    a_s = jnp.dot(hg, as16[...], preferred_element_type=jnp.float32)
    a_d = jnp.dot(hg, ad16[...], preferred_element_type=jnp.float32)
    hgs_out[:, :D] = hg
    hgs_out[:, D:] = a_s
    td_out[...] = a_d
    cs_out[0, 0, :] = jnp.max(a_s, axis=0)
    cs_out[0, 1, :] = jnp.max(a_d, axis=0)


def _bn_gat(hpre, stats, resid, g, be, wg, as16, ad16):
    return pl.pallas_call(
        _bn_gat_body,
        grid=(NBLK,),
        in_specs=[_blk((BLK, D)), _full((NBLK, 2, D)), _blk((BLK, D)),
                  _full((1, D)), _full((1, D)), _full((D, D)),
                  _full((D, 16)), _full((D, 16))],
        out_specs=(_blk((BLK, DW)), _blk((BLK, 16)),
                   pl.BlockSpec((1, 2, 16), lambda i: (i, 0, 0))),
        out_shape=(jax.ShapeDtypeStruct((NPAD, DW), jnp.float32),
                   jax.ShapeDtypeStruct((NPAD, 16), jnp.float32),
                   jax.ShapeDtypeStruct((NBLK, 2, 16), jnp.float32)),
    )(hpre, stats, resid, g, be, wg, as16, ad16)


def _head_body(nd0, nd1, hgs, td, c16, bg, wc1, bc1, wc2, bc2,
               emb_out, log_out):
    asum = hgs[:, D:] + td[...]
    wself = jnp.exp(jnp.maximum(asum, 0.2 * asum) - c16[...])
    rden = 1.0 / (nd0[:, D:] + nd1[:, D:] + wself + 1e-16)
    emb = jnp.zeros((BLK, D_OUT), jnp.float32)
    for head in range(HEADS):
        sl = slice(head * D_OUT, (head + 1) * D_OUT)
        blk = (nd0[:, sl] + nd1[:, sl] + hgs[:, sl] * wself[:, head:head + 1])
        emb = emb + blk * rden[:, head:head + 1]
    emb = emb * (1.0 / HEADS) + bg[...]
    z = jnp.maximum(
        jnp.dot(emb, wc1[...], preferred_element_type=jnp.float32) + bc1[...],
        0.0)
    log_out[...] = (jnp.dot(z, wc2[...], preferred_element_type=jnp.float32)
                    + bc2[...])
    emb_out[...] = emb


def _head(nd0, nd1, hgs, td, c16, bg, wc1, bc1, wc2, bc2):
    return pl.pallas_call(
        _head_body,
        grid=(NBLK,),
        in_specs=[_blk((BLK, DW)), _blk((BLK, DW)), _blk((BLK, DW)),
                  _blk((BLK, 16)), _full((1, 16)), _full((1, D_OUT)),
                  _full((D_OUT, 64)), _full((1, 64)), _full((64, 1)),
                  _full((1, 1))],
        out_specs=(_blk((BLK, D_OUT)), _blk((BLK, 1))),
        out_shape=(jax.ShapeDtypeStruct((N, D_OUT), jnp.float32),
                   jax.ShapeDtypeStruct((N, 1), jnp.float32)),
    )(nd0, nd1, hgs, td, c16, bg, wc1, bc1, wc2, bc2)


def kernel(x, edge_index, W_l1, b_l1, W_r1, g1, be1, W_l2, b_l2, W_r2, g2,
           be2, W_g, att_s, att_d, b_g, Wc1, bc1, Wc2, bc2):
    pad = N + (jnp.arange(EPAD - E, dtype=jnp.int32) % 16)
    src = jnp.concatenate([edge_index[0], pad]).reshape(NTILES * NCHS, CHS)
    dst = jnp.concatenate([edge_index[1], pad]).reshape(NTILES * NCHS, CHS)
    idx = jnp.stack([src, dst], axis=1)

    padg = N + (jnp.arange(EPADG - E, dtype=jnp.int32) % 16)
    srcg = jnp.concatenate([edge_index[0], padg]).reshape(NTILES * NCHG, CHG)
    dstg = jnp.concatenate([edge_index[1], padg]).reshape(NTILES * NCHG, CHG)
    idxg = jnp.stack([srcg, dstg], axis=1)

    mask = (jnp.arange(16)[None, :] ==
            (jnp.arange(D) // D_OUT)[:, None]).astype(jnp.float32)
    as16 = mask * att_s.reshape(D, 1)
    ad16 = mask * att_d.reshape(D, 1)

    z144 = jnp.zeros((NPAD, DW), jnp.float32)
    z128 = jnp.zeros((NPAD, D), jnp.float32)

    b_l1r = b_l1.reshape(1, D)
    g1r = g1.reshape(1, D)
    be1r = be1.reshape(1, D)
    b_l2r = b_l2.reshape(1, D)
    g2r = g2.reshape(1, D)
    be2r = be2.reshape(1, D)
    bgr = b_g.reshape(1, D_OUT)
    bc1r = bc1.reshape(1, 64)
    bc2r = bc2.reshape(1, 1)

    acc1 = _make_sage_sc(DW)(_xa(x), idx, z144)
    hpre1, st1, invc = _sage_dense(acc1[0], acc1[1], x, W_l1, b_l1r, W_r1)
    h = _bn_finish(hpre1, st1, x, g1r, be1r)

    acc2 = _make_sage_sc(D)(h, idx, z128)
    hpre2, st2 = _sage_dense2(acc2[0], acc2[1], invc, h, W_l2, b_l2r, W_r2)
    hgs, td, cs = _bn_gat(hpre2, st2, h, g2r, be2r, W_g, as16, ad16)

    c16 = jnp.maximum(jnp.max(cs[:, 0, :], axis=0, keepdims=True)
                      + jnp.max(cs[:, 1, :], axis=0, keepdims=True), 0.0)
    nd = _make_gat_sc()(hgs, td, c16, idxg, z144)
    emb, logits = _head(nd[0], nd[1], hgs, td, c16, bgr, Wc1, bc1r, Wc2,
                        bc2r)
    return (emb, logits)


# speedup vs baseline: 54.6405x; 1.0129x over previous
import functools

import jax
import jax.numpy as jnp
from jax import lax
from jax.experimental import pallas as pl
from jax.experimental.pallas import tpu as pltpu
from jax.experimental.pallas import tpu_sc as plsc

N = 10000
E = 320000
D = 128
DW = D + 16
HEADS = 4
D_OUT = 32

NCORES = 2
NSUB = 16
NTILES = NCORES * NSUB
CHS = 128
NCHS = 80
NPAIR = NCHS // 2
EPAD = NTILES * NCHS * CHS
NPAD = N + 16
RPT = NPAD // NSUB

BLK = 2000
NBLK = N // BLK

CHG = 120
NCHG = 84
NPAIRG = NCHG // 2
EPADG = NTILES * NCHG * CHG

_SC_PARAMS = pltpu.CompilerParams(use_tc_tiling_on_sc=False,
                                  needs_layout_passes=False)


@functools.lru_cache(maxsize=None)
def _mesh():
    return plsc.VectorSubcoreMesh(
        core_axis_name="c", subcore_axis_name="s",
        num_cores=NCORES, num_subcores=NSUB)


def _wait(src, dst, sem):
    pltpu.make_async_copy(src, dst, sem).wait()


@functools.lru_cache(maxsize=None)
def _make_sage_sc(width):
    @functools.partial(
        pl.kernel,
        out_type=jax.ShapeDtypeStruct((NCORES, NPAD, width), jnp.float32),
        mesh=_mesh(),
        scratch_types=[
            pltpu.VMEM_SHARED((NPAD, width), jnp.float32),
            pltpu.VMEM((2, CHS), jnp.int32),
            pltpu.VMEM((2, CHS), jnp.int32),
            pltpu.VMEM((CHS, width), jnp.float32),
            pltpu.VMEM((CHS, width), jnp.float32),
            pltpu.SemaphoreType.DMA,
            pltpu.SemaphoreType.DMA,
            pltpu.SemaphoreType.DMA,
            pltpu.SemaphoreType.DMA,
        ],
        compiler_params=_SC_PARAMS,
    )
    def sage_sc(tab_hbm, idx_hbm, zero_hbm, out_hbm,
                acc, ixa, ixb, bufa, bufb, sga, sgb, ssa, ssb):
        c = lax.axis_index("c")
        s = lax.axis_index("s")
        base = (c * NSUB + s) * NCHS
        r0 = s * RPT
        pltpu.sync_copy(zero_hbm.at[pl.ds(r0, RPT)], acc.at[pl.ds(r0, RPT)])
        plsc.subcore_barrier()

        pltpu.sync_copy(idx_hbm.at[base], ixa)
        pltpu.sync_copy(idx_hbm.at[base + 1], ixb)
        pltpu.async_copy(tab_hbm.at[ixa.at[0]], bufa, sga)
        pltpu.async_copy(tab_hbm.at[ixb.at[0]], bufb, sgb)

        @pl.loop(0, NPAIR - 1)
        def _(k):
            _wait(tab_hbm.at[ixa.at[0]], bufa, sga)
            pltpu.async_copy(bufa, acc.at[ixa.at[1]], ssa, add=True)
            _wait(tab_hbm.at[ixb.at[0]], bufb, sgb)
            pltpu.async_copy(bufb, acc.at[ixb.at[1]], ssb, add=True)
            _wait(bufa, acc.at[ixa.at[1]], ssa)
            pltpu.sync_copy(idx_hbm.at[base + 2 * k + 2], ixa)
            pltpu.async_copy(tab_hbm.at[ixa.at[0]], bufa, sga)
            _wait(bufb, acc.at[ixb.at[1]], ssb)
            pltpu.sync_copy(idx_hbm.at[base + 2 * k + 3], ixb)
            pltpu.async_copy(tab_hbm.at[ixb.at[0]], bufb, sgb)

        _wait(tab_hbm.at[ixa.at[0]], bufa, sga)
        pltpu.async_copy(bufa, acc.at[ixa.at[1]], ssa, add=True)
        _wait(tab_hbm.at[ixb.at[0]], bufb, sgb)
        pltpu.async_copy(bufb, acc.at[ixb.at[1]], ssb, add=True)
        _wait(bufa, acc.at[ixa.at[1]], ssa)
        _wait(bufb, acc.at[ixb.at[1]], ssb)

        plsc.subcore_barrier()
        pltpu.sync_copy(acc.at[pl.ds(r0, RPT)], out_hbm.at[c, pl.ds(r0, RPT)])

    return sage_sc


@functools.lru_cache(maxsize=None)
def _make_gat_sc():
  @functools.partial(
    pl.kernel,
    out_type=jax.ShapeDtypeStruct((NCORES, NPAD, DW), jnp.float32),
    mesh=_mesh(),
    scratch_types=[
        pltpu.VMEM_SHARED((NPAD, DW), jnp.float32),
        pltpu.VMEM((2, CHG), jnp.int32),
        pltpu.VMEM((2, CHG), jnp.int32),
        pltpu.VMEM((CHG, DW), jnp.float32),
        pltpu.VMEM((CHG, DW), jnp.float32),
        pltpu.VMEM((CHG, 16), jnp.float32),
        pltpu.VMEM((CHG, 16), jnp.float32),
        pltpu.VMEM((1, 16), jnp.float32),
        pltpu.SemaphoreType.DMA,
        pltpu.SemaphoreType.DMA,
        pltpu.SemaphoreType.DMA,
        pltpu.SemaphoreType.DMA,
        pltpu.SemaphoreType.DMA,
        pltpu.SemaphoreType.DMA,
    ],
    compiler_params=_SC_PARAMS,
  )
  def _gat_sc(hgs_hbm, tabd_hbm, c16_hbm, idx_hbm, zero_hbm, out_hbm,
              acc, ixa, ixb, ha, hb, da, db, cscr,
              sga, sgb, sda, sdb, ssa, ssb):
    c = lax.axis_index("c")
    s = lax.axis_index("s")
    base = (c * NSUB + s) * NCHG
    r0 = s * RPT
    pltpu.sync_copy(zero_hbm.at[pl.ds(r0, RPT)], acc.at[pl.ds(r0, RPT)])
    pltpu.sync_copy(c16_hbm, cscr)
    plsc.subcore_barrier()
    cvec = cscr[0, :]

    def compute(hbuf, dbuf):
        @pl.loop(0, CHG, step=4)
        def _(e0):
            for u in range(4):
                e = e0 + u
                a = hbuf[e, pl.ds(D, 16)] + dbuf[e, :]
                hbuf[e, pl.ds(D, 16)] = jnp.exp(
                    jnp.maximum(a, 0.2 * a) - cvec)

        @pl.loop(0, CHG, step=4)
        def _(i0):
            for u in range(4):
                i = i0 + u
                w16 = hbuf[i, pl.ds(D, 16)]
                for head in range(HEADS):
                    wb = lax.gather(
                        w16, jnp.full((16, 1), head, jnp.int32),
                        lax.GatherDimensionNumbers(
                            offset_dims=(), collapsed_slice_dims=(0,),
                            start_index_map=(0,)),
                        (1,), mode=lax.GatherScatterMode.PROMISE_IN_BOUNDS)
                    for half in range(2):
                        col = (head * 2 + half) * 16
                        hbuf[i, pl.ds(col, 16)] = (
                            hbuf[i, pl.ds(col, 16)] * wb)

    pltpu.sync_copy(idx_hbm.at[base], ixa)
    pltpu.sync_copy(idx_hbm.at[base + 1], ixb)
    pltpu.async_copy(hgs_hbm.at[ixa.at[0]], ha, sga)
    pltpu.async_copy(tabd_hbm.at[ixa.at[1]], da, sda)
    pltpu.async_copy(hgs_hbm.at[ixb.at[0]], hb, sgb)
    pltpu.async_copy(tabd_hbm.at[ixb.at[1]], db, sdb)

    @pl.loop(0, NPAIRG - 1)
    def _(k):
        _wait(hgs_hbm.at[ixa.at[0]], ha, sga)
        _wait(tabd_hbm.at[ixa.at[1]], da, sda)
        compute(ha, da)
        pltpu.async_copy(ha, acc.at[ixa.at[1]], ssa, add=True)
        _wait(hgs_hbm.at[ixb.at[0]], hb, sgb)
        _wait(tabd_hbm.at[ixb.at[1]], db, sdb)
        compute(hb, db)
        pltpu.async_copy(hb, acc.at[ixb.at[1]], ssb, add=True)
        _wait(ha, acc.at[ixa.at[1]], ssa)
        pltpu.sync_copy(idx_hbm.at[base + 2 * k + 2], ixa)
        pltpu.async_copy(hgs_hbm.at[ixa.at[0]], ha, sga)
        pltpu.async_copy(tabd_hbm.at[ixa.at[1]], da, sda)
        _wait(hb, acc.at[ixb.at[1]], ssb)
        pltpu.sync_copy(idx_hbm.at[base + 2 * k + 3], ixb)
        pltpu.async_copy(hgs_hbm.at[ixb.at[0]], hb, sgb)
        pltpu.async_copy(tabd_hbm.at[ixb.at[1]], db, sdb)

    _wait(hgs_hbm.at[ixa.at[0]], ha, sga)
    _wait(tabd_hbm.at[ixa.at[1]], da, sda)
    compute(ha, da)
    pltpu.async_copy(ha, acc.at[ixa.at[1]], ssa, add=True)
    _wait(hgs_hbm.at[ixb.at[0]], hb, sgb)
    _wait(tabd_hbm.at[ixb.at[1]], db, sdb)
    compute(hb, db)
    pltpu.async_copy(hb, acc.at[ixb.at[1]], ssb, add=True)
    _wait(ha, acc.at[ixa.at[1]], ssa)
    _wait(hb, acc.at[ixb.at[1]], ssb)

    plsc.subcore_barrier()
    pltpu.sync_copy(acc.at[pl.ds(r0, RPT)], out_hbm.at[c, pl.ds(r0, RPT)])

  return _gat_sc


def _blk(shape):
    nd = len(shape)
    return pl.BlockSpec((BLK,) + shape[1:], lambda i: (i,) + (0,) * (nd - 1))


def _full(shape):
    nd = len(shape)
    return pl.BlockSpec(shape, lambda i: (0,) * nd)


def _sage_dense_body(a0, a1, xr, wl, bl, wr, hpre_out, stats_out, invc_out):
    su = a0[:, :D] + a1[:, :D]
    cnt = a0[:, D:D + 1] + a1[:, D:D + 1]
    invc = 1.0 / jnp.maximum(cnt, 1.0)
    mean = su * invc
    h = (jnp.dot(mean, wl[...], preferred_element_type=jnp.float32) + bl[...]
         + jnp.dot(xr[...], wr[...], preferred_element_type=jnp.float32))
    hpre_out[...] = h
    stats_out[0, 0, :] = jnp.sum(h, axis=0)
    stats_out[0, 1, :] = jnp.sum(h * h, axis=0)
    invc_out[...] = jnp.broadcast_to(invc, (BLK, 8))


def _sage_dense(a0, a1, xr, wl, bl, wr):
    return pl.pallas_call(
        _sage_dense_body,
        grid=(NBLK,),
        in_specs=[_blk((BLK, DW)), _blk((BLK, DW)), _blk((BLK, D)),
                  _full((D, D)), _full((1, D)), _full((D, D))],
        out_specs=(_blk((BLK, D)),
                   pl.BlockSpec((1, 2, D), lambda i: (i, 0, 0)),
                   _blk((BLK, 8))),
        out_shape=(jax.ShapeDtypeStruct((N, D), jnp.float32),
                   jax.ShapeDtypeStruct((NBLK, 2, D), jnp.float32),
                   jax.ShapeDtypeStruct((N, 8), jnp.float32)),
    )(a0, a1, xr, wl, bl, wr)


def _sage_dense2_body(a0, a1, invc, xr, wl, bl, wr, hpre_out, stats_out):
    su = a0[...] + a1[...]
    mean = su * invc[:, 0:1]
    h = (jnp.dot(mean, wl[...], preferred_element_type=jnp.float32) + bl[...]
         + jnp.dot(xr[...], wr[...], preferred_element_type=jnp.float32))
    hpre_out[...] = h
    stats_out[0, 0, :] = jnp.sum(h, axis=0)
    stats_out[0, 1, :] = jnp.sum(h * h, axis=0)


def _sage_dense2(a0, a1, invc, xr, wl, bl, wr):
    return pl.pallas_call(
        _sage_dense2_body,
        grid=(NBLK,),
        in_specs=[_blk((BLK, D)), _blk((BLK, D)), _blk((BLK, 8)),
                  _blk((BLK, D)), _full((D, D)), _full((1, D)),
                  _full((D, D))],
        out_specs=(_blk((BLK, D)),
                   pl.BlockSpec((1, 2, D), lambda i: (i, 0, 0))),
        out_shape=(jax.ShapeDtypeStruct((N, D), jnp.float32),
                   jax.ShapeDtypeStruct((NBLK, 2, D), jnp.float32)),
    )(a0, a1, invc, xr, wl, bl, wr)


def _bn_finish_body(hpre, stats, resid, g, be, h_out):
    mu = jnp.sum(stats[:, 0, :], axis=0, keepdims=True) * (1.0 / N)
    ex2 = jnp.sum(stats[:, 1, :], axis=0, keepdims=True) * (1.0 / N)
    var = ex2 - mu * mu
    h = g[...] * (hpre[...] - mu) * jax.lax.rsqrt(var + 1e-5) + be[...]
    h_out[...] = jnp.maximum(h + resid[...], 0.0)


def _bn_finish(hpre, stats, resid, g, be):
    return pl.pallas_call(
        _bn_finish_body,
        grid=(NBLK,),
        in_specs=[_blk((BLK, D)), _full((NBLK, 2, D)), _blk((BLK, D)),
                  _full((1, D)), _full((1, D))],
        out_specs=_blk((BLK, D)),
        out_shape=jax.ShapeDtypeStruct((NPAD, D), jnp.float32),
    )(hpre, stats, resid, g, be)


def _xa_body(xr, xa_out):
    xa_out[:, :D] = xr[...]
    xa_out[:, D:D + 1] = jnp.ones((BLK, 1), jnp.float32)
    xa_out[:, D + 1:] = jnp.zeros((BLK, 15), jnp.float32)


def _xa(xr):
    return pl.pallas_call(
        _xa_body,
        grid=(NBLK,),
        in_specs=[_blk((BLK, D))],
        out_specs=_blk((BLK, DW)),
        out_shape=jax.ShapeDtypeStruct((NPAD, DW), jnp.float32),
    )(xr)


def _bn_gat_body(hpre, stats, resid, g, be, wg, as16, ad16,
                 hgs_out, td_out, cs_out):
    mu = jnp.sum(stats[:, 0, :], axis=0, keepdims=True) * (1.0 / N)
    ex2 = jnp.sum(stats[:, 1, :], axis=0, keepdims=True) * (1.0 / N)
    var = ex2 - mu * mu
    h = g[...] * (hpre[...] - mu) * jax.lax.rsqrt(var + 1e-5) + be[...]
    h2 = jnp.maximum(h + resid[...], 0.0)
    hg = jnp.dot(h2, wg[...], preferred_element_type=jnp.float32)
    a_s = jnp.dot(hg, as16[...], preferred_element_type=jnp.float32)
    a_d = jnp.dot(hg, ad16[...], preferred_element_type=jnp.float32)
    hgs_out[:, :D] = hg
    hgs_out[:, D:] = a_s
    td_out[...] = a_d
    cs_out[0, 0, :] = jnp.max(a_s, axis=0)
    cs_out[0, 1, :] = jnp.max(a_d, axis=0)


def _bn_gat(hpre, stats, resid, g, be, wg, as16, ad16):
    return pl.pallas_call(
        _bn_gat_body,
        grid=(NBLK,),
        in_specs=[_blk((BLK, D)), _full((NBLK, 2, D)), _blk((BLK, D)),
                  _full((1, D)), _full((1, D)), _full((D, D)),
                  _full((D, 16)), _full((D, 16))],
        out_specs=(_blk((BLK, DW)), _blk((BLK, 16)),
                   pl.BlockSpec((1, 2, 16), lambda i: (i, 0, 0))),
        out_shape=(jax.ShapeDtypeStruct((NPAD, DW), jnp.float32),
                   jax.ShapeDtypeStruct((NPAD, 16), jnp.float32),
                   jax.ShapeDtypeStruct((NBLK, 2, 16), jnp.float32)),
    )(hpre, stats, resid, g, be, wg, as16, ad16)


def _head_body(nd0, nd1, hgs, td, c16, bg, wc1, bc1, wc2, bc2,
               emb_out, log_out):
    asum = hgs[:, D:] + td[...]
    wself = jnp.exp(jnp.maximum(asum, 0.2 * asum) - c16[...])
    rden = 1.0 / (nd0[:, D:] + nd1[:, D:] + wself + 1e-16)
    emb = jnp.zeros((BLK, D_OUT), jnp.float32)
    for head in range(HEADS):
        sl = slice(head * D_OUT, (head + 1) * D_OUT)
        blk = (nd0[:, sl] + nd1[:, sl] + hgs[:, sl] * wself[:, head:head + 1])
        emb = emb + blk * rden[:, head:head + 1]
    emb = emb * (1.0 / HEADS) + bg[...]
    z = jnp.maximum(
        jnp.dot(emb, wc1[...], preferred_element_type=jnp.float32) + bc1[...],
        0.0)
    log_out[...] = (jnp.dot(z, wc2[...], preferred_element_type=jnp.float32)
                    + bc2[...])
    emb_out[...] = emb


def _head(nd0, nd1, hgs, td, c16, bg, wc1, bc1, wc2, bc2):
    return pl.pallas_call(
        _head_body,
        grid=(NBLK,),
        in_specs=[_blk((BLK, DW)), _blk((BLK, DW)), _blk((BLK, DW)),
                  _blk((BLK, 16)), _full((1, 16)), _full((1, D_OUT)),
                  _full((D_OUT, 64)), _full((1, 64)), _full((64, 1)),
                  _full((1, 1))],
        out_specs=(_blk((BLK, D_OUT)), _blk((BLK, 1))),
        out_shape=(jax.ShapeDtypeStruct((N, D_OUT), jnp.float32),
                   jax.ShapeDtypeStruct((N, 1), jnp.float32)),
    )(nd0, nd1, hgs, td, c16, bg, wc1, bc1, wc2, bc2)


def kernel(x, edge_index, W_l1, b_l1, W_r1, g1, be1, W_l2, b_l2, W_r2, g2,
           be2, W_g, att_s, att_d, b_g, Wc1, bc1, Wc2, bc2):
    pad = N + (jnp.arange(EPAD - E, dtype=jnp.int32) % 16)
    src = jnp.concatenate([edge_index[0], pad]).reshape(NTILES * NCHS, CHS)
    dst = jnp.concatenate([edge_index[1], pad]).reshape(NTILES * NCHS, CHS)
    idx = jnp.stack([src, dst], axis=1)

    padg = N + (jnp.arange(EPADG - E, dtype=jnp.int32) % 16)
    srcg = jnp.concatenate([edge_index[0], padg]).reshape(NTILES * NCHG, CHG)
    dstg = jnp.concatenate([edge_index[1], padg]).reshape(NTILES * NCHG, CHG)
    idxg = jnp.stack([srcg, dstg], axis=1)

    mask = (jnp.arange(16)[None, :] ==
            (jnp.arange(D) // D_OUT)[:, None]).astype(jnp.float32)
    as16 = mask * att_s.reshape(D, 1)
    ad16 = mask * att_d.reshape(D, 1)

    z144 = jnp.zeros((NPAD, DW), jnp.float32)
    z128 = jnp.zeros((NPAD, D), jnp.float32)

    b_l1r = b_l1.reshape(1, D)
    g1r = g1.reshape(1, D)
    be1r = be1.reshape(1, D)
    b_l2r = b_l2.reshape(1, D)
    g2r = g2.reshape(1, D)
    be2r = be2.reshape(1, D)
    bgr = b_g.reshape(1, D_OUT)
    bc1r = bc1.reshape(1, 64)
    bc2r = bc2.reshape(1, 1)

    acc1 = _make_sage_sc(DW)(_xa(x), idx, z144)
    hpre1, st1, invc = _sage_dense(acc1[0], acc1[1], x, W_l1, b_l1r, W_r1)
    h = _bn_finish(hpre1, st1, x, g1r, be1r)

    acc2 = _make_sage_sc(D)(h, idx, z128)
    hpre2, st2 = _sage_dense2(acc2[0], acc2[1], invc, h, W_l2, b_l2r, W_r2)
    hgs, td, cs = _bn_gat(hpre2, st2, h, g2r, be2r, W_g, as16, ad16)

    c16 = jnp.maximum(jnp.max(cs[:, 0, :], axis=0, keepdims=True)
                      + jnp.max(cs[:, 1, :], axis=0, keepdims=True), 0.0)
    nd = _make_gat_sc()(hgs, td, c16, idxg, z144)
    emb, logits = _head(nd[0], nd[1], hgs, td, c16, bgr, Wc1, bc1r, Wc2,
                        bc2r)
    return (emb, logits)
